# Initial kernel scaffold; baseline (speedup 1.0000x reference)
import jax
import jax.numpy as jnp
from jax.experimental import pallas as pl


def kernel(x, edge_index, edge_label_index, W1_l, W1_r, b1, W2_l, W2_r, b2):
    raise NotImplementedError("write your pallas kernel here")


# speedup vs baseline: 4.9742x; 4.9742x over previous
import functools

import jax
import jax.numpy as jnp
from jax import lax
from jax.experimental import pallas as pl
from jax.experimental.pallas import tpu as pltpu
from jax.experimental.pallas import tpu_sc as plsc

N_NODES = 10000
IN_DIM = 128
HID_DIM = 128
OUT_DIM = 64
N_EDGES = 320000
N_LABEL = 100000

NC = 2
NS = 16
NW = NC * NS

NPAD = 10240
RPT = NPAD // NS
EPT = N_EDGES // NW
EC = 200
ECH = EPT // EC
NLPAD = 102400
LPT = NLPAD // NW
LC = 400
LCH = LPT // LC

_mesh = plsc.VectorSubcoreMesh(core_axis_name="c", subcore_axis_name="s")


def _make_agg(dim, with_cnt):
  out_type = [jax.ShapeDtypeStruct((NC * NPAD, dim), jnp.float32)]
  scratch = [
      pltpu.VMEM_SHARED((NPAD, dim), jnp.float32),
      pltpu.VMEM((EC,), jnp.int32),
      pltpu.VMEM((EC,), jnp.int32),
      pltpu.VMEM((EC, dim), jnp.float32),
      pltpu.SemaphoreType.DMA,
  ]
  if with_cnt:
    out_type.append(jax.ShapeDtypeStruct((NC * NPAD,), jnp.float32))
    scratch += [
        pltpu.VMEM_SHARED((NPAD,), jnp.float32),
        pltpu.VMEM((EC + 8,), jnp.float32),
        pltpu.VMEM((RPT,), jnp.float32),
    ]

  nvec = dim // 16

  def body(feat, src_h, dst_h, *rest):
    if with_cnt:
      (part_o, cnt_o, acc_sh, src_v, dst_v, rows_v, sem,
       cnt_sh, ones_v, zrow_v) = rest
    else:
      (part_o, acc_sh, src_v, dst_v, rows_v, sem) = rest
    c = lax.axis_index("c")
    s = lax.axis_index("s")
    wid = c * NS + s
    base_r = s * RPT

    def zrow(i, _):
      r = i // nvec
      k = i - r * nvec
      rows_v[r, pl.ds(k * 16, 16)] = jnp.zeros((16,), jnp.float32)
      return 0
    lax.fori_loop(0, EC * nvec, zrow, 0)
    for j0 in range(0, RPT, EC):
      n = min(EC, RPT - j0)
      pltpu.sync_copy(rows_v.at[pl.ds(0, n)], acc_sh.at[pl.ds(base_r + j0, n)])

    if with_cnt:
      def fill1(i, _):
        ones_v[pl.ds(i * 16, 16)] = jnp.ones((16,), jnp.float32)
        return 0
      lax.fori_loop(0, (EC + 8) // 16, fill1, 0)
      def fill0(i, _):
        zrow_v[pl.ds(i * 16, 16)] = jnp.zeros((16,), jnp.float32)
        return 0
      lax.fori_loop(0, RPT // 16, fill0, 0)
      pltpu.sync_copy(zrow_v, cnt_sh.at[pl.ds(base_r, RPT)])

    plsc.subcore_barrier()

    ebase = wid * EPT
    def step(g, _):
      b = ebase + g * EC
      pltpu.sync_copy(src_h.at[pl.ds(b, EC)], src_v)
      pltpu.sync_copy(dst_h.at[pl.ds(b, EC)], dst_v)
      pltpu.async_copy(feat.at[src_v], rows_v, sem).wait()
      pltpu.sync_copy(rows_v, acc_sh.at[dst_v], add=True)
      if with_cnt:
        pltpu.sync_copy(ones_v.at[pl.ds(0, EC)], cnt_sh.at[dst_v], add=True)
      return 0
    lax.fori_loop(0, ECH, step, 0)

    plsc.subcore_barrier()

    obase = c * NPAD + base_r
    pltpu.sync_copy(acc_sh.at[pl.ds(base_r, RPT)], part_o.at[pl.ds(obase, RPT)])
    if with_cnt:
      pltpu.sync_copy(cnt_sh.at[pl.ds(base_r, RPT)], cnt_o.at[pl.ds(obase, RPT)])

  return pl.kernel(body, out_type=out_type, mesh=_mesh, scratch_types=scratch)


_agg_cnt = _make_agg(IN_DIM, True)
_agg_plain = _make_agg(HID_DIM, False)


def _tc_layer(part, cnt, feat, w_l, w_r, b, relu, hpad=None):
  n, din = feat.shape
  h = w_l.shape[0]
  hpad = h if hpad is None else hpad
  blk = 1000
  grid = (n // blk,)

  def body(part_ref, cnt_ref, feat_ref, wl_ref, wr_ref, b_ref, out_ref):
    p = part_ref[0] + part_ref[1]
    cn = jnp.maximum(cnt_ref[0] + cnt_ref[1], 1.0)
    agg = p / cn
    o = lax.dot_general(agg, wl_ref[...], (((1,), (1,)), ((), ())),
                        preferred_element_type=jnp.float32)
    o = o + lax.dot_general(feat_ref[...], wr_ref[...], (((1,), (1,)), ((), ())),
                            preferred_element_type=jnp.float32)
    o = o + b_ref[...][None, :]
    if relu:
      o = jnp.maximum(o, 0.0)
    if hpad > h:
      o = jnp.concatenate([o, jnp.zeros((o.shape[0], hpad - h), jnp.float32)],
                          axis=1)
    out_ref[...] = o

  return pl.pallas_call(
      body,
      grid=grid,
      in_specs=[
          pl.BlockSpec((2, blk, din), lambda i: (0, i, 0)),
          pl.BlockSpec((2, blk, 1), lambda i: (0, i, 0)),
          pl.BlockSpec((blk, din), lambda i: (i, 0)),
          pl.BlockSpec((h, din), lambda i: (0, 0)),
          pl.BlockSpec((h, din), lambda i: (0, 0)),
          pl.BlockSpec((h,), lambda i: (0,)),
      ],
      out_specs=pl.BlockSpec((blk, hpad), lambda i: (i, 0)),
      out_shape=jax.ShapeDtypeStruct((n, hpad), jnp.float32),
  )(part, cnt, feat, w_l, w_r, b)


def _make_decode(dim):
  out_type = [jax.ShapeDtypeStruct((NLPAD, dim), jnp.float32)]
  scratch = [
      pltpu.VMEM((LC,), jnp.int32),
      pltpu.VMEM((LC,), jnp.int32),
      pltpu.VMEM((LC, dim), jnp.float32),
      pltpu.VMEM((LC, dim), jnp.float32),
      pltpu.SemaphoreType.DMA,
  ]

  def body(z_h, ls_h, ld_h, prod_o, si_v, di_v, srows_v, drows_v, sem):
    c = lax.axis_index("c")
    s = lax.axis_index("s")
    wid = c * NS + s
    tbase = wid * LPT

    def chunk(g, _):
      b = tbase + g * LC
      pltpu.sync_copy(ls_h.at[pl.ds(b, LC)], si_v)
      pltpu.sync_copy(ld_h.at[pl.ds(b, LC)], di_v)
      pltpu.async_copy(z_h.at[si_v], srows_v, sem).wait()
      pltpu.async_copy(z_h.at[di_v], drows_v, sem).wait()

      def prow(r, _):
        for k in range(dim // 16):
          srows_v[r, pl.ds(k * 16, 16)] = (
              srows_v[r, pl.ds(k * 16, 16)] * drows_v[r, pl.ds(k * 16, 16)])
        return 0
      lax.fori_loop(0, LC, prow, 0)
      pltpu.sync_copy(srows_v, prod_o.at[pl.ds(b, LC)])
      return 0
    lax.fori_loop(0, LCH, chunk, 0)

  return pl.kernel(body, out_type=out_type, mesh=_mesh, scratch_types=scratch)


_decode = _make_decode(128)


def _rowsum(prod, dim):
  blk = 2048

  def body(p_ref, o_ref):
    o_ref[...] = jnp.sum(p_ref[...], axis=1, keepdims=True)

  return pl.pallas_call(
      body,
      grid=(NLPAD // blk,),
      in_specs=[pl.BlockSpec((blk, dim), lambda i: (i, 0))],
      out_specs=pl.BlockSpec((blk, 1), lambda i: (i, 0)),
      out_shape=jax.ShapeDtypeStruct((NLPAD, 1), jnp.float32),
  )(prod)


def kernel(x, edge_index, edge_label_index, W1_l, W1_r, b1, W2_l, W2_r, b2):
  x = x.astype(jnp.float32)
  src = jnp.asarray(edge_index[0], jnp.int32)
  dst = jnp.asarray(edge_index[1], jnp.int32)
  pad = jnp.zeros((NLPAD - N_LABEL,), jnp.int32)
  ls = jnp.concatenate([jnp.asarray(edge_label_index[0], jnp.int32), pad])
  ld = jnp.concatenate([jnp.asarray(edge_label_index[1], jnp.int32), pad])

  part1, cnt = _agg_cnt(x, src, dst)
  part1 = part1.reshape(NC, NPAD, IN_DIM)
  cnt2 = cnt.reshape(NC, NPAD, 1)
  h = _tc_layer(part1, cnt2, x, W1_l, W1_r, b1, relu=True)
  (part2,) = _agg_plain(h, src, dst)
  part2 = part2.reshape(NC, NPAD, HID_DIM)
  z = _tc_layer(part2, cnt2, h, W2_l, W2_r, b2, relu=False, hpad=128)
  (prod,) = _decode(z, ls, ld)
  dots = _rowsum(prod, 128)
  return dots[:N_LABEL, 0]


# speedup vs baseline: 5.2417x; 1.0538x over previous
import functools

import jax
import jax.numpy as jnp
from jax import lax
from jax.experimental import pallas as pl
from jax.experimental.pallas import tpu as pltpu
from jax.experimental.pallas import tpu_sc as plsc

N_NODES = 10000
IN_DIM = 128
HID_DIM = 128
OUT_DIM = 64
N_EDGES = 320000
N_LABEL = 100000

NC = 2
NS = 16
NW = NC * NS

NPAD = 10240
RPT = NPAD // NS
EPT = N_EDGES // NW
ECA = 120
ECB = 80
EPAIR = ECA + ECB
NPAIR = EPT // EPAIR
NLPAD = 102400
LPT = NLPAD // NW
LC = 400
LCH = LPT // LC

_mesh = plsc.VectorSubcoreMesh(core_axis_name="c", subcore_axis_name="s")


def _make_agg(dim, with_cnt):
  out_type = [jax.ShapeDtypeStruct((NC * NPAD, dim), jnp.float32)]
  scratch = [
      pltpu.VMEM_SHARED((NPAD, dim), jnp.float32),
      pltpu.VMEM((ECA,), jnp.int32),
      pltpu.VMEM((ECB,), jnp.int32),
      pltpu.VMEM((ECA,), jnp.int32),
      pltpu.VMEM((ECB,), jnp.int32),
      pltpu.VMEM((ECA, dim), jnp.float32),
      pltpu.VMEM((ECB, dim), jnp.float32),
  ] + [pltpu.SemaphoreType.DMA] * 8
  if with_cnt:
    out_type.append(jax.ShapeDtypeStruct((NC * NPAD,), jnp.float32))
    scratch += [
        pltpu.VMEM_SHARED((NPAD,), jnp.float32),
        pltpu.VMEM((ECA + 8,), jnp.float32),
        pltpu.VMEM((RPT,), jnp.float32),
        pltpu.SemaphoreType.DMA,
        pltpu.SemaphoreType.DMA,
    ]

  nvec = dim // 16
  szs = (ECA, ECB)
  offs = (0, ECA)

  def body(feat, src_h, dst_h, *rest):
    if with_cnt:
      (part_o, cnt_o, acc_sh, srcA, srcB, dstA, dstB, rowsA, rowsB,
       gsemA, gsemB, ssemA, ssemB, sisemA, sisemB, disemA, disemB,
       cnt_sh, ones_v, zrow_v, csemA, csemB) = rest
    else:
      (part_o, acc_sh, srcA, srcB, dstA, dstB, rowsA, rowsB,
       gsemA, gsemB, ssemA, ssemB, sisemA, sisemB, disemA, disemB) = rest
      cnt_sh = ones_v = zrow_v = csemA = csemB = None
    src_v = (srcA, srcB)
    dst_v = (dstA, dstB)
    rows_v = (rowsA, rowsB)
    gsem = (gsemA, gsemB)
    ssem = (ssemA, ssemB)
    sisem = (sisemA, sisemB)
    disem = (disemA, disemB)
    csem = (csemA, csemB)

    c = lax.axis_index("c")
    s = lax.axis_index("s")
    wid = c * NS + s
    base_r = s * RPT
    ebase = wid * EPT

    def zrow(i, _):
      r = i // nvec
      k = i - r * nvec
      rowsA[r, pl.ds(k * 16, 16)] = jnp.zeros((16,), jnp.float32)
      return 0
    lax.fori_loop(0, ECA * nvec, zrow, 0)
    for j0 in range(0, RPT, ECA):
      n = min(ECA, RPT - j0)
      pltpu.sync_copy(rowsA.at[pl.ds(0, n)], acc_sh.at[pl.ds(base_r + j0, n)])

    if with_cnt:
      def fill1(i, _):
        ones_v[pl.ds(i * 16, 16)] = jnp.ones((16,), jnp.float32)
        return 0
      lax.fori_loop(0, (ECA + 8) // 16, fill1, 0)
      def fill0(i, _):
        zrow_v[pl.ds(i * 16, 16)] = jnp.zeros((16,), jnp.float32)
        return 0
      lax.fori_loop(0, RPT // 16, fill0, 0)
      pltpu.sync_copy(zrow_v, cnt_sh.at[pl.ds(base_r, RPT)])

    def issue_idx_src(g, b):
      off = ebase + g * EPAIR + offs[b]
      pltpu.async_copy(src_h.at[pl.ds(off, szs[b])], src_v[b], sisem[b])

    def issue_idx_dst(g, b):
      off = ebase + g * EPAIR + offs[b]
      pltpu.async_copy(dst_h.at[pl.ds(off, szs[b])], dst_v[b], disem[b])

    def wait_idx(b):
      pltpu.make_async_copy(src_h.at[pl.ds(0, szs[b])], src_v[b], sisem[b]).wait()
      pltpu.make_async_copy(dst_h.at[pl.ds(0, szs[b])], dst_v[b], disem[b]).wait()

    def issue_gather(b):
      pltpu.async_copy(feat.at[src_v[b]], rows_v[b], gsem[b])

    def wait_gather(b):
      pltpu.make_async_copy(feat.at[src_v[b]], rows_v[b], gsem[b]).wait()

    def issue_scatter(b):
      pltpu.async_copy(rows_v[b], acc_sh.at[dst_v[b]], ssem[b], add=True)
      if with_cnt:
        pltpu.async_copy(ones_v.at[pl.ds(0, szs[b])], cnt_sh.at[dst_v[b]],
                         csem[b], add=True)

    def wait_scatter(b):
      pltpu.make_async_copy(rows_v[b], acc_sh.at[dst_v[b]], ssem[b]).wait()
      if with_cnt:
        pltpu.make_async_copy(ones_v.at[pl.ds(0, szs[b])], cnt_sh.at[dst_v[b]],
                              csem[b]).wait()

    issue_idx_src(0, 0)
    issue_idx_dst(0, 0)
    plsc.subcore_barrier()
    wait_idx(0)
    issue_gather(0)

    def pair(g, _):
      wait_gather(0)
      issue_idx_src(g, 1)
      issue_scatter(0)

      @pl.when(g > 0)
      def _():
        wait_scatter(1)
      issue_idx_dst(g, 1)
      wait_idx(1)
      issue_gather(1)

      wait_gather(1)

      @pl.when(g < NPAIR - 1)
      def _():
        issue_idx_src(g + 1, 0)
      issue_scatter(1)
      wait_scatter(0)

      @pl.when(g < NPAIR - 1)
      def _():
        issue_idx_dst(g + 1, 0)
        wait_idx(0)
        issue_gather(0)
      return 0
    lax.fori_loop(0, NPAIR, pair, 0)

    wait_scatter(1)
    plsc.subcore_barrier()

    obase = c * NPAD + base_r
    pltpu.sync_copy(acc_sh.at[pl.ds(base_r, RPT)], part_o.at[pl.ds(obase, RPT)])
    if with_cnt:
      pltpu.sync_copy(cnt_sh.at[pl.ds(base_r, RPT)], cnt_o.at[pl.ds(obase, RPT)])

  return pl.kernel(body, out_type=out_type, mesh=_mesh, scratch_types=scratch)


_agg_cnt = _make_agg(IN_DIM, True)
_agg_plain = _make_agg(HID_DIM, False)


def _tc_layer(part, cnt, feat, w_l, w_r, b, relu, hpad=None):
  n, din = feat.shape
  h = w_l.shape[0]
  hpad = h if hpad is None else hpad
  blk = 1000
  grid = (n // blk,)

  def body(part_ref, cnt_ref, feat_ref, wl_ref, wr_ref, b_ref, out_ref):
    p = part_ref[0] + part_ref[1]
    cn = jnp.maximum(cnt_ref[0] + cnt_ref[1], 1.0)
    agg = p / cn
    o = lax.dot_general(agg, wl_ref[...], (((1,), (1,)), ((), ())),
                        preferred_element_type=jnp.float32)
    o = o + lax.dot_general(feat_ref[...], wr_ref[...], (((1,), (1,)), ((), ())),
                            preferred_element_type=jnp.float32)
    o = o + b_ref[...][None, :]
    if relu:
      o = jnp.maximum(o, 0.0)
    if hpad > h:
      o = jnp.concatenate([o, jnp.zeros((o.shape[0], hpad - h), jnp.float32)],
                          axis=1)
    out_ref[...] = o

  return pl.pallas_call(
      body,
      grid=grid,
      in_specs=[
          pl.BlockSpec((2, blk, din), lambda i: (0, i, 0)),
          pl.BlockSpec((2, blk, 1), lambda i: (0, i, 0)),
          pl.BlockSpec((blk, din), lambda i: (i, 0)),
          pl.BlockSpec((h, din), lambda i: (0, 0)),
          pl.BlockSpec((h, din), lambda i: (0, 0)),
          pl.BlockSpec((h,), lambda i: (0,)),
      ],
      out_specs=pl.BlockSpec((blk, hpad), lambda i: (i, 0)),
      out_shape=jax.ShapeDtypeStruct((n, hpad), jnp.float32),
  )(part, cnt, feat, w_l, w_r, b)


def _make_decode(dim):
  out_type = [jax.ShapeDtypeStruct((NLPAD, dim), jnp.float32)]
  scratch = [
      pltpu.VMEM((LC,), jnp.int32),
      pltpu.VMEM((LC,), jnp.int32),
      pltpu.VMEM((LC, dim), jnp.float32),
      pltpu.VMEM((LC, dim), jnp.float32),
      pltpu.SemaphoreType.DMA,
  ]

  def body(z_h, ls_h, ld_h, prod_o, si_v, di_v, srows_v, drows_v, sem):
    c = lax.axis_index("c")
    s = lax.axis_index("s")
    wid = c * NS + s
    tbase = wid * LPT

    def chunk(g, _):
      b = tbase + g * LC
      pltpu.sync_copy(ls_h.at[pl.ds(b, LC)], si_v)
      pltpu.sync_copy(ld_h.at[pl.ds(b, LC)], di_v)
      pltpu.async_copy(z_h.at[si_v], srows_v, sem).wait()
      pltpu.async_copy(z_h.at[di_v], drows_v, sem).wait()

      def prow(r, _):
        for k in range(dim // 16):
          srows_v[r, pl.ds(k * 16, 16)] = (
              srows_v[r, pl.ds(k * 16, 16)] * drows_v[r, pl.ds(k * 16, 16)])
        return 0
      lax.fori_loop(0, LC, prow, 0)
      pltpu.sync_copy(srows_v, prod_o.at[pl.ds(b, LC)])
      return 0
    lax.fori_loop(0, LCH, chunk, 0)

  return pl.kernel(body, out_type=out_type, mesh=_mesh, scratch_types=scratch)


_decode = _make_decode(128)


def _rowsum(prod, dim):
  blk = 2048

  def body(p_ref, o_ref):
    o_ref[...] = jnp.sum(p_ref[...], axis=1, keepdims=True)

  return pl.pallas_call(
      body,
      grid=(NLPAD // blk,),
      in_specs=[pl.BlockSpec((blk, dim), lambda i: (i, 0))],
      out_specs=pl.BlockSpec((blk, 1), lambda i: (i, 0)),
      out_shape=jax.ShapeDtypeStruct((NLPAD, 1), jnp.float32),
  )(prod)


def kernel(x, edge_index, edge_label_index, W1_l, W1_r, b1, W2_l, W2_r, b2):
  x = x.astype(jnp.float32)
  src = jnp.asarray(edge_index[0], jnp.int32)
  dst = jnp.asarray(edge_index[1], jnp.int32)
  pad = jnp.zeros((NLPAD - N_LABEL,), jnp.int32)
  ls = jnp.concatenate([jnp.asarray(edge_label_index[0], jnp.int32), pad])
  ld = jnp.concatenate([jnp.asarray(edge_label_index[1], jnp.int32), pad])

  part1, cnt = _agg_cnt(x, src, dst)
  part1 = part1.reshape(NC, NPAD, IN_DIM)
  cnt2 = cnt.reshape(NC, NPAD, 1)
  h = _tc_layer(part1, cnt2, x, W1_l, W1_r, b1, relu=True)
  (part2,) = _agg_plain(h, src, dst)
  part2 = part2.reshape(NC, NPAD, HID_DIM)
  z = _tc_layer(part2, cnt2, h, W2_l, W2_r, b2, relu=False, hpad=128)
  (prod,) = _decode(z, ls, ld)
  dots = _rowsum(prod, 128)
  return dots[:N_LABEL, 0]


# speedup vs baseline: 5.4331x; 1.0365x over previous
import functools

import jax
import jax.numpy as jnp
from jax import lax
from jax.experimental import pallas as pl
from jax.experimental.pallas import tpu as pltpu
from jax.experimental.pallas import tpu_sc as plsc

N_NODES = 10000
IN_DIM = 128
HID_DIM = 128
OUT_DIM = 64
N_EDGES = 320000
N_LABEL = 100000

NC = 2
NS = 16
NW = NC * NS

NPAD = 10240
RPT = NPAD // NS
EPT = N_EDGES // NW
ECA = 120
ECB = 80
EPAIR = ECA + ECB
NPAIR = EPT // EPAIR
NLPAD = 102400
LPT = NLPAD // NW
LC = 160
LCH = LPT // LC

_mesh = plsc.VectorSubcoreMesh(core_axis_name="c", subcore_axis_name="s")


def _make_agg(dim, with_cnt):
  out_type = [jax.ShapeDtypeStruct((NC * NPAD, dim), jnp.float32)]
  scratch = [
      pltpu.VMEM_SHARED((NPAD, dim), jnp.float32),
      pltpu.VMEM((ECA,), jnp.int32),
      pltpu.VMEM((ECB,), jnp.int32),
      pltpu.VMEM((ECA,), jnp.int32),
      pltpu.VMEM((ECB,), jnp.int32),
      pltpu.VMEM((ECA, dim), jnp.float32),
      pltpu.VMEM((ECB, dim), jnp.float32),
  ] + [pltpu.SemaphoreType.DMA] * 8
  if with_cnt:
    out_type.append(jax.ShapeDtypeStruct((NC * NPAD,), jnp.float32))
    scratch += [
        pltpu.VMEM_SHARED((NPAD,), jnp.float32),
        pltpu.VMEM((ECA + 8,), jnp.float32),
        pltpu.VMEM((RPT,), jnp.float32),
        pltpu.SemaphoreType.DMA,
        pltpu.SemaphoreType.DMA,
    ]

  nvec = dim // 16
  szs = (ECA, ECB)
  offs = (0, ECA)

  def body(feat, src_h, dst_h, *rest):
    if with_cnt:
      (part_o, cnt_o, acc_sh, srcA, srcB, dstA, dstB, rowsA, rowsB,
       gsemA, gsemB, ssemA, ssemB, sisemA, sisemB, disemA, disemB,
       cnt_sh, ones_v, zrow_v, csemA, csemB) = rest
    else:
      (part_o, acc_sh, srcA, srcB, dstA, dstB, rowsA, rowsB,
       gsemA, gsemB, ssemA, ssemB, sisemA, sisemB, disemA, disemB) = rest
      cnt_sh = ones_v = zrow_v = csemA = csemB = None
    src_v = (srcA, srcB)
    dst_v = (dstA, dstB)
    rows_v = (rowsA, rowsB)
    gsem = (gsemA, gsemB)
    ssem = (ssemA, ssemB)
    sisem = (sisemA, sisemB)
    disem = (disemA, disemB)
    csem = (csemA, csemB)

    c = lax.axis_index("c")
    s = lax.axis_index("s")
    wid = c * NS + s
    base_r = s * RPT
    ebase = wid * EPT

    def zrow(i, _):
      r = i // nvec
      k = i - r * nvec
      rowsA[r, pl.ds(k * 16, 16)] = jnp.zeros((16,), jnp.float32)
      return 0
    lax.fori_loop(0, ECA * nvec, zrow, 0)
    for j0 in range(0, RPT, ECA):
      n = min(ECA, RPT - j0)
      pltpu.sync_copy(rowsA.at[pl.ds(0, n)], acc_sh.at[pl.ds(base_r + j0, n)])

    if with_cnt:
      def fill1(i, _):
        ones_v[pl.ds(i * 16, 16)] = jnp.ones((16,), jnp.float32)
        return 0
      lax.fori_loop(0, (ECA + 8) // 16, fill1, 0)
      def fill0(i, _):
        zrow_v[pl.ds(i * 16, 16)] = jnp.zeros((16,), jnp.float32)
        return 0
      lax.fori_loop(0, RPT // 16, fill0, 0)
      pltpu.sync_copy(zrow_v, cnt_sh.at[pl.ds(base_r, RPT)])

    def issue_idx_src(g, b):
      off = ebase + g * EPAIR + offs[b]
      pltpu.async_copy(src_h.at[pl.ds(off, szs[b])], src_v[b], sisem[b])

    def issue_idx_dst(g, b):
      off = ebase + g * EPAIR + offs[b]
      pltpu.async_copy(dst_h.at[pl.ds(off, szs[b])], dst_v[b], disem[b])

    def wait_idx(b):
      pltpu.make_async_copy(src_h.at[pl.ds(0, szs[b])], src_v[b], sisem[b]).wait()
      pltpu.make_async_copy(dst_h.at[pl.ds(0, szs[b])], dst_v[b], disem[b]).wait()

    def issue_gather(b):
      pltpu.async_copy(feat.at[src_v[b]], rows_v[b], gsem[b])

    def wait_gather(b):
      pltpu.make_async_copy(feat.at[src_v[b]], rows_v[b], gsem[b]).wait()

    def issue_scatter(b):
      pltpu.async_copy(rows_v[b], acc_sh.at[dst_v[b]], ssem[b], add=True)
      if with_cnt:
        pltpu.async_copy(ones_v.at[pl.ds(0, szs[b])], cnt_sh.at[dst_v[b]],
                         csem[b], add=True)

    def wait_scatter(b):
      pltpu.make_async_copy(rows_v[b], acc_sh.at[dst_v[b]], ssem[b]).wait()
      if with_cnt:
        pltpu.make_async_copy(ones_v.at[pl.ds(0, szs[b])], cnt_sh.at[dst_v[b]],
                              csem[b]).wait()

    issue_idx_src(0, 0)
    issue_idx_dst(0, 0)
    plsc.subcore_barrier()
    wait_idx(0)
    issue_gather(0)

    def pair(g, _):
      wait_gather(0)
      issue_idx_src(g, 1)
      issue_scatter(0)

      @pl.when(g > 0)
      def _():
        wait_scatter(1)
      issue_idx_dst(g, 1)
      wait_idx(1)
      issue_gather(1)

      wait_gather(1)

      @pl.when(g < NPAIR - 1)
      def _():
        issue_idx_src(g + 1, 0)
      issue_scatter(1)
      wait_scatter(0)

      @pl.when(g < NPAIR - 1)
      def _():
        issue_idx_dst(g + 1, 0)
        wait_idx(0)
        issue_gather(0)
      return 0
    lax.fori_loop(0, NPAIR, pair, 0)

    wait_scatter(1)
    plsc.subcore_barrier()

    obase = c * NPAD + base_r
    pltpu.sync_copy(acc_sh.at[pl.ds(base_r, RPT)], part_o.at[pl.ds(obase, RPT)])
    if with_cnt:
      pltpu.sync_copy(cnt_sh.at[pl.ds(base_r, RPT)], cnt_o.at[pl.ds(obase, RPT)])

  return pl.kernel(body, out_type=out_type, mesh=_mesh, scratch_types=scratch)


_agg_cnt = _make_agg(IN_DIM, True)
_agg_plain = _make_agg(HID_DIM, False)


def _tc_layer(part, cnt, feat, w_l, w_r, b, relu, hpad=None):
  n, din = feat.shape
  h = w_l.shape[0]
  hpad = h if hpad is None else hpad
  blk = 1000
  grid = (n // blk,)

  def body(part_ref, cnt_ref, feat_ref, wl_ref, wr_ref, b_ref, out_ref):
    p = part_ref[0] + part_ref[1]
    cn = jnp.maximum(cnt_ref[0] + cnt_ref[1], 1.0)
    agg = p / cn
    o = lax.dot_general(agg, wl_ref[...], (((1,), (1,)), ((), ())),
                        preferred_element_type=jnp.float32)
    o = o + lax.dot_general(feat_ref[...], wr_ref[...], (((1,), (1,)), ((), ())),
                            preferred_element_type=jnp.float32)
    o = o + b_ref[...][None, :]
    if relu:
      o = jnp.maximum(o, 0.0)
    if hpad > h:
      o = jnp.concatenate([o, jnp.zeros((o.shape[0], hpad - h), jnp.float32)],
                          axis=1)
    out_ref[...] = o

  return pl.pallas_call(
      body,
      grid=grid,
      in_specs=[
          pl.BlockSpec((2, blk, din), lambda i: (0, i, 0)),
          pl.BlockSpec((2, blk, 1), lambda i: (0, i, 0)),
          pl.BlockSpec((blk, din), lambda i: (i, 0)),
          pl.BlockSpec((h, din), lambda i: (0, 0)),
          pl.BlockSpec((h, din), lambda i: (0, 0)),
          pl.BlockSpec((h,), lambda i: (0,)),
      ],
      out_specs=pl.BlockSpec((blk, hpad), lambda i: (i, 0)),
      out_shape=jax.ShapeDtypeStruct((n, hpad), jnp.float32),
  )(part, cnt, feat, w_l, w_r, b)


def _make_decode(gdim, pdim):
  out_type = [jax.ShapeDtypeStruct((NLPAD, pdim), jnp.float32)]
  scratch = (
      [pltpu.VMEM((LC,), jnp.int32)] * 4 +
      [pltpu.VMEM((LC, gdim), jnp.float32)] * 4 +
      [pltpu.VMEM((LC, pdim), jnp.float32)] * 2 +
      [pltpu.SemaphoreType.DMA] * 10
  )

  def body(z_h, ls_h, ld_h, prod_o,
           siA, siB, diA, diB, sA, sB, dA, dB, pA, pB,
           gsA, gsB, gdA, gdB, isA, isB, idA, idB, wsA, wsB):
    si_v = (siA, siB)
    di_v = (diA, diB)
    srows = (sA, sB)
    drows = (dA, dB)
    prod_v = (pA, pB)
    gssem = (gsA, gsB)
    gdsem = (gdA, gdB)
    issem = (isA, isB)
    idsem = (idA, idB)
    wsem = (wsA, wsB)

    c = lax.axis_index("c")
    s = lax.axis_index("s")
    wid = c * NS + s
    tbase = wid * LPT

    def issue_idx(g, b):
      off = tbase + g * LC
      pltpu.async_copy(ls_h.at[pl.ds(off, LC)], si_v[b], issem[b])
      pltpu.async_copy(ld_h.at[pl.ds(off, LC)], di_v[b], idsem[b])

    def wait_idx(b):
      pltpu.make_async_copy(ls_h.at[pl.ds(0, LC)], si_v[b], issem[b]).wait()
      pltpu.make_async_copy(ld_h.at[pl.ds(0, LC)], di_v[b], idsem[b]).wait()

    def issue_gather(b):
      pltpu.async_copy(z_h.at[si_v[b]], srows[b], gssem[b])
      pltpu.async_copy(z_h.at[di_v[b]], drows[b], gdsem[b])

    def wait_gather(b):
      pltpu.make_async_copy(z_h.at[si_v[b]], srows[b], gssem[b]).wait()
      pltpu.make_async_copy(z_h.at[di_v[b]], drows[b], gdsem[b]).wait()

    def issue_write(g, b):
      pltpu.async_copy(prod_v[b], prod_o.at[pl.ds(tbase + g * LC, LC)], wsem[b])

    def wait_write(b):
      pltpu.make_async_copy(prod_v[b], prod_o.at[pl.ds(0, LC)], wsem[b]).wait()

    issue_idx(0, 0)
    wait_idx(0)
    issue_gather(0)

    def pair(g, _):
      for b in (0, 1):
        cidx = 2 * g + b
        o = 1 - b
        wait_gather(b)

        @pl.when(cidx + 1 < LCH)
        def _():
          issue_idx(cidx + 1, o)

        @pl.when(cidx > 0)
        def _():
          wait_write(o)

        @pl.when(cidx + 1 < LCH)
        def _():
          wait_idx(o)
          issue_gather(o)

        def prow(r, _):
          for k in range(pdim // 16):
            prod_v[b][r, pl.ds(k * 16, 16)] = (
                srows[b][r, pl.ds(k * 16, 16)] * drows[b][r, pl.ds(k * 16, 16)])
          return 0
        lax.fori_loop(0, LC, prow, 0)
        issue_write(cidx, b)
      return 0
    lax.fori_loop(0, LCH // 2, pair, 0)
    wait_write(1)
    plsc.subcore_barrier()

  return pl.kernel(body, out_type=out_type, mesh=_mesh, scratch_types=scratch)


_decode = _make_decode(128, OUT_DIM)


def _rowsum(prod, dim):
  blk = 2048

  def body(p_ref, o_ref):
    o_ref[...] = jnp.sum(p_ref[...], axis=1, keepdims=True)

  return pl.pallas_call(
      body,
      grid=(NLPAD // blk,),
      in_specs=[pl.BlockSpec((blk, dim), lambda i: (i, 0))],
      out_specs=pl.BlockSpec((blk, 1), lambda i: (i, 0)),
      out_shape=jax.ShapeDtypeStruct((NLPAD, 1), jnp.float32),
  )(prod)


def kernel(x, edge_index, edge_label_index, W1_l, W1_r, b1, W2_l, W2_r, b2):
  x = x.astype(jnp.float32)
  src = jnp.asarray(edge_index[0], jnp.int32)
  dst = jnp.asarray(edge_index[1], jnp.int32)
  pad = jnp.zeros((NLPAD - N_LABEL,), jnp.int32)
  ls = jnp.concatenate([jnp.asarray(edge_label_index[0], jnp.int32), pad])
  ld = jnp.concatenate([jnp.asarray(edge_label_index[1], jnp.int32), pad])

  part1, cnt = _agg_cnt(x, src, dst)
  part1 = part1.reshape(NC, NPAD, IN_DIM)
  cnt2 = cnt.reshape(NC, NPAD, 1)
  h = _tc_layer(part1, cnt2, x, W1_l, W1_r, b1, relu=True)
  (part2,) = _agg_plain(h, src, dst)
  part2 = part2.reshape(NC, NPAD, HID_DIM)
  z = _tc_layer(part2, cnt2, h, W2_l, W2_r, b2, relu=False, hpad=128)
  (prod,) = _decode(z, ls, ld)
  dots = _rowsum(prod, OUT_DIM)
  return dots[:N_LABEL, 0]


# speedup vs baseline: 5.6721x; 1.0440x over previous
import functools

import jax
import jax.numpy as jnp
from jax import lax
from jax.experimental import pallas as pl
from jax.experimental.pallas import tpu as pltpu
from jax.experimental.pallas import tpu_sc as plsc

N_NODES = 10000
IN_DIM = 128
HID_DIM = 128
OUT_DIM = 64
N_EDGES = 320000
N_LABEL = 100000

NC = 2
NS = 16
NW = NC * NS

NPAD = 10112
RPT = NPAD // NS
CPAD = 10240
CRPT = CPAD // NS
EPT = N_EDGES // NW
ECU = 176
NCHUNK = 56
TAIL = EPT - NCHUNK * ECU
NLPAD = 102400
LPT = NLPAD // NW
LC = 160
LCH = LPT // LC

_mesh = plsc.VectorSubcoreMesh(core_axis_name="c", subcore_axis_name="s")


def _make_agg(dim, with_cnt):
  out_type = [jax.ShapeDtypeStruct((NC * NPAD, dim), jnp.float32)]
  scratch = (
      [pltpu.VMEM_SHARED((NPAD, dim), jnp.float32)] +
      [pltpu.VMEM((ECU,), jnp.int32)] * 4 +
      [pltpu.VMEM((ECU,), jnp.int32)] * 4 +
      [pltpu.VMEM((ECU, dim), jnp.float32)] * 2 +
      [pltpu.VMEM((TAIL,), jnp.int32)] +
      [pltpu.SemaphoreType.DMA] * 12
  )
  if with_cnt:
    out_type.append(jax.ShapeDtypeStruct((NC * CPAD,), jnp.float32))
    scratch += [
        pltpu.VMEM_SHARED((CPAD,), jnp.float32),
        pltpu.VMEM((192,), jnp.float32),
        pltpu.VMEM((CRPT,), jnp.float32),
        pltpu.SemaphoreType.DMA,
        pltpu.SemaphoreType.DMA,
    ]

  nvec = dim // 16

  def body(feat, src_h, dst_h, *rest):
    if with_cnt:
      (part_o, cnt_o, acc_sh, s0, s1, s2, s3, d0, d1, d2, d3, rowsA, rowsB,
       tail_d, si0, si1, si2, si3, di0, di1, di2, di3, gsA, gsB, scA, scB,
       cnt_sh, ones_v, zrow_v, csA, csB) = rest
    else:
      (part_o, acc_sh, s0, s1, s2, s3, d0, d1, d2, d3, rowsA, rowsB,
       tail_d, si0, si1, si2, si3, di0, di1, di2, di3, gsA, gsB, scA, scB) = rest
      cnt_sh = ones_v = zrow_v = csA = csB = None
    src_v = (s0, s1, s2, s3)
    dst_v = (d0, d1, d2, d3)
    rows_v = (rowsA, rowsB)
    sisem = (si0, si1, si2, si3)
    disem = (di0, di1, di2, di3)
    gsem = (gsA, gsB)
    ssem = (scA, scB)
    csem = (csA, csB)

    c = lax.axis_index("c")
    s = lax.axis_index("s")
    wid = c * NS + s
    base_r = s * RPT
    ebase = wid * EPT

    def zrow(i, _):
      r = i // nvec
      k = i - r * nvec
      rowsA[r, pl.ds(k * 16, 16)] = jnp.zeros((16,), jnp.float32)
      return 0
    lax.fori_loop(0, ECU * nvec, zrow, 0)
    for j0 in range(0, RPT, ECU):
      n = min(ECU, RPT - j0)
      pltpu.sync_copy(rowsA.at[pl.ds(0, n)], acc_sh.at[pl.ds(base_r + j0, n)])

    if with_cnt:
      def fill1(i, _):
        ones_v[pl.ds(i * 16, 16)] = jnp.ones((16,), jnp.float32)
        return 0
      lax.fori_loop(0, 192 // 16, fill1, 0)
      def fill0(i, _):
        zrow_v[pl.ds(i * 16, 16)] = jnp.zeros((16,), jnp.float32)
        return 0
      lax.fori_loop(0, CRPT // 16, fill0, 0)
      pltpu.sync_copy(zrow_v, cnt_sh.at[pl.ds(s * CRPT, CRPT)])

    def issue_idx(cidx, q):
      off = ebase + cidx * ECU
      pltpu.async_copy(src_h.at[pl.ds(off, ECU)], src_v[q], sisem[q])
      pltpu.async_copy(dst_h.at[pl.ds(off, ECU)], dst_v[q], disem[q])

    def wait_idx(q):
      pltpu.make_async_copy(src_h.at[pl.ds(0, ECU)], src_v[q], sisem[q]).wait()
      pltpu.make_async_copy(dst_h.at[pl.ds(0, ECU)], dst_v[q], disem[q]).wait()

    def issue_gather(q, b):
      pltpu.async_copy(feat.at[src_v[q]], rows_v[b], gsem[b])

    def wait_gather(q, b):
      pltpu.make_async_copy(feat.at[src_v[q]], rows_v[b], gsem[b]).wait()

    def issue_scatter(q, b):
      pltpu.async_copy(rows_v[b], acc_sh.at[dst_v[q]], ssem[b], add=True)
      if with_cnt:
        pltpu.async_copy(ones_v.at[pl.ds(0, ECU)], cnt_sh.at[dst_v[q]],
                         csem[b], add=True)

    def wait_scatter(q, b):
      pltpu.make_async_copy(rows_v[b], acc_sh.at[dst_v[q]], ssem[b]).wait()
      if with_cnt:
        pltpu.make_async_copy(ones_v.at[pl.ds(0, ECU)], cnt_sh.at[dst_v[q]],
                              csem[b]).wait()

    issue_idx(0, 0)
    issue_idx(1, 1)
    plsc.subcore_barrier()
    wait_idx(0)
    issue_gather(0, 0)

    def quad(g, _):
      for c4 in range(4):
        b = c4 % 2
        o = 1 - b
        qn = (c4 + 1) % 4
        qp = (c4 + 2) % 4
        qo = (c4 + 3) % 4

        wait_gather(c4, b)

        def prefetch(gg=g, qq=qp, cc4=c4):
          issue_idx(4 * gg + cc4 + 2, qq)
        if c4 < 2:
          prefetch()
        else:
          pl.when(g < NCHUNK // 4 - 1)(prefetch)

        issue_scatter(c4, b)

        def drain(qq=qo, bb=o):
          wait_scatter(qq, bb)
        if c4 > 0:
          drain()
        else:
          pl.when(g > 0)(drain)

        def nxt(qq=qn, bb=o):
          wait_idx(qq)
          issue_gather(qq, bb)
        if c4 < 3:
          nxt()
        else:
          pl.when(g < NCHUNK // 4 - 1)(nxt)
      return 0
    lax.fori_loop(0, NCHUNK // 4, quad, 0)

    wait_scatter(3, 1)

    toff = ebase + NCHUNK * ECU
    pltpu.async_copy(src_h.at[pl.ds(toff, TAIL)], s0.at[pl.ds(0, TAIL)], si0)
    pltpu.async_copy(dst_h.at[pl.ds(toff, TAIL)], tail_d, di0)
    pltpu.make_async_copy(src_h.at[pl.ds(0, TAIL)], s0.at[pl.ds(0, TAIL)],
                          si0).wait()
    pltpu.make_async_copy(dst_h.at[pl.ds(0, TAIL)], tail_d, di0).wait()
    pltpu.async_copy(feat.at[s0.at[pl.ds(0, TAIL)]], rowsA.at[pl.ds(0, TAIL)],
                     gsA).wait()
    pltpu.async_copy(rowsA.at[pl.ds(0, TAIL)], acc_sh.at[tail_d], scA,
                     add=True).wait()
    if with_cnt:
      pltpu.async_copy(ones_v.at[pl.ds(0, TAIL)], cnt_sh.at[tail_d], csA,
                       add=True).wait()

    plsc.subcore_barrier()

    obase = c * NPAD + base_r
    pltpu.sync_copy(acc_sh.at[pl.ds(base_r, RPT)], part_o.at[pl.ds(obase, RPT)])
    if with_cnt:
      pltpu.sync_copy(cnt_sh.at[pl.ds(s * CRPT, CRPT)],
                      cnt_o.at[pl.ds(c * CPAD + s * CRPT, CRPT)])

  return pl.kernel(body, out_type=out_type, mesh=_mesh, scratch_types=scratch)


_agg_cnt = _make_agg(IN_DIM, True)
_agg_plain = _make_agg(HID_DIM, False)


def _tc_layer(part, cnt, feat, w_l, w_r, b, relu, hpad=None):
  n, din = feat.shape
  h = w_l.shape[0]
  hpad = h if hpad is None else hpad
  blk = 1000
  grid = (n // blk,)

  def body(part_ref, cnt_ref, feat_ref, wl_ref, wr_ref, b_ref, out_ref):
    p = part_ref[0] + part_ref[1]
    cn = jnp.maximum(cnt_ref[0] + cnt_ref[1], 1.0)
    agg = p / cn
    o = lax.dot_general(agg, wl_ref[...], (((1,), (1,)), ((), ())),
                        preferred_element_type=jnp.float32)
    o = o + lax.dot_general(feat_ref[...], wr_ref[...], (((1,), (1,)), ((), ())),
                            preferred_element_type=jnp.float32)
    o = o + b_ref[...][None, :]
    if relu:
      o = jnp.maximum(o, 0.0)
    if hpad > h:
      o = jnp.concatenate([o, jnp.zeros((o.shape[0], hpad - h), jnp.float32)],
                          axis=1)
    out_ref[...] = o

  return pl.pallas_call(
      body,
      grid=grid,
      in_specs=[
          pl.BlockSpec((2, blk, din), lambda i: (0, i, 0)),
          pl.BlockSpec((2, blk, 1), lambda i: (0, i, 0)),
          pl.BlockSpec((blk, din), lambda i: (i, 0)),
          pl.BlockSpec((h, din), lambda i: (0, 0)),
          pl.BlockSpec((h, din), lambda i: (0, 0)),
          pl.BlockSpec((h,), lambda i: (0,)),
      ],
      out_specs=pl.BlockSpec((blk, hpad), lambda i: (i, 0)),
      out_shape=jax.ShapeDtypeStruct((n, hpad), jnp.float32),
  )(part, cnt, feat, w_l, w_r, b)


def _make_decode(gdim, pdim):
  out_type = [jax.ShapeDtypeStruct((NLPAD, pdim), jnp.float32)]
  scratch = (
      [pltpu.VMEM((LC,), jnp.int32)] * 4 +
      [pltpu.VMEM((LC, gdim), jnp.float32)] * 4 +
      [pltpu.VMEM((LC, pdim), jnp.float32)] * 2 +
      [pltpu.SemaphoreType.DMA] * 10
  )

  def body(z_h, ls_h, ld_h, prod_o,
           siA, siB, diA, diB, sA, sB, dA, dB, pA, pB,
           gsA, gsB, gdA, gdB, isA, isB, idA, idB, wsA, wsB):
    si_v = (siA, siB)
    di_v = (diA, diB)
    srows = (sA, sB)
    drows = (dA, dB)
    prod_v = (pA, pB)
    gssem = (gsA, gsB)
    gdsem = (gdA, gdB)
    issem = (isA, isB)
    idsem = (idA, idB)
    wsem = (wsA, wsB)

    c = lax.axis_index("c")
    s = lax.axis_index("s")
    wid = c * NS + s
    tbase = wid * LPT

    def issue_idx(g, b):
      off = tbase + g * LC
      pltpu.async_copy(ls_h.at[pl.ds(off, LC)], si_v[b], issem[b])
      pltpu.async_copy(ld_h.at[pl.ds(off, LC)], di_v[b], idsem[b])

    def wait_idx(b):
      pltpu.make_async_copy(ls_h.at[pl.ds(0, LC)], si_v[b], issem[b]).wait()
      pltpu.make_async_copy(ld_h.at[pl.ds(0, LC)], di_v[b], idsem[b]).wait()

    def issue_gather(b):
      pltpu.async_copy(z_h.at[si_v[b]], srows[b], gssem[b])
      pltpu.async_copy(z_h.at[di_v[b]], drows[b], gdsem[b])

    def wait_gather(b):
      pltpu.make_async_copy(z_h.at[si_v[b]], srows[b], gssem[b]).wait()
      pltpu.make_async_copy(z_h.at[di_v[b]], drows[b], gdsem[b]).wait()

    def issue_write(g, b):
      pltpu.async_copy(prod_v[b], prod_o.at[pl.ds(tbase + g * LC, LC)], wsem[b])

    def wait_write(b):
      pltpu.make_async_copy(prod_v[b], prod_o.at[pl.ds(0, LC)], wsem[b]).wait()

    issue_idx(0, 0)
    wait_idx(0)
    issue_gather(0)

    def pair(g, _):
      for b in (0, 1):
        cidx = 2 * g + b
        o = 1 - b
        wait_gather(b)

        @pl.when(cidx + 1 < LCH)
        def _():
          issue_idx(cidx + 1, o)

        @pl.when(cidx > 0)
        def _():
          wait_write(o)

        @pl.when(cidx + 1 < LCH)
        def _():
          wait_idx(o)
          issue_gather(o)

        def prow(r, _):
          for k in range(pdim // 16):
            prod_v[b][r, pl.ds(k * 16, 16)] = (
                srows[b][r, pl.ds(k * 16, 16)] * drows[b][r, pl.ds(k * 16, 16)])
          return 0
        lax.fori_loop(0, LC, prow, 0)
        issue_write(cidx, b)
      return 0
    lax.fori_loop(0, LCH // 2, pair, 0)
    wait_write(1)
    plsc.subcore_barrier()

  return pl.kernel(body, out_type=out_type, mesh=_mesh, scratch_types=scratch)


_decode = _make_decode(128, OUT_DIM)


def _rowsum(prod, dim):
  blk = 2048

  def body(p_ref, o_ref):
    o_ref[...] = jnp.sum(p_ref[...], axis=1, keepdims=True)

  return pl.pallas_call(
      body,
      grid=(NLPAD // blk,),
      in_specs=[pl.BlockSpec((blk, dim), lambda i: (i, 0))],
      out_specs=pl.BlockSpec((blk, 1), lambda i: (i, 0)),
      out_shape=jax.ShapeDtypeStruct((NLPAD, 1), jnp.float32),
  )(prod)


def kernel(x, edge_index, edge_label_index, W1_l, W1_r, b1, W2_l, W2_r, b2):
  x = x.astype(jnp.float32)
  src = jnp.asarray(edge_index[0], jnp.int32)
  dst = jnp.asarray(edge_index[1], jnp.int32)
  pad = jnp.zeros((NLPAD - N_LABEL,), jnp.int32)
  ls = jnp.concatenate([jnp.asarray(edge_label_index[0], jnp.int32), pad])
  ld = jnp.concatenate([jnp.asarray(edge_label_index[1], jnp.int32), pad])

  part1, cnt = _agg_cnt(x, src, dst)
  part1 = part1.reshape(NC, NPAD, IN_DIM)
  cnt2 = cnt.reshape(NC, CPAD, 1)
  h = _tc_layer(part1, cnt2, x, W1_l, W1_r, b1, relu=True)
  (part2,) = _agg_plain(h, src, dst)
  part2 = part2.reshape(NC, NPAD, HID_DIM)
  z = _tc_layer(part2, cnt2, h, W2_l, W2_r, b2, relu=False, hpad=128)
  (prod,) = _decode(z, ls, ld)
  dots = _rowsum(prod, OUT_DIM)
  return dots[:N_LABEL, 0]


# speedup vs baseline: 7.8015x; 1.3754x over previous
import functools

import jax
import jax.numpy as jnp
from jax import lax
from jax.experimental import pallas as pl
from jax.experimental.pallas import tpu as pltpu
from jax.experimental.pallas import tpu_sc as plsc

N_NODES = 10000
IN_DIM = 128
HID_DIM = 128
OUT_DIM = 64
N_EDGES = 320000
N_LABEL = 100000

NC = 2
NS = 16
NW = NC * NS

NPAD = 10112
RPT = NPAD // NS
CPAD = 10240
CRPT = CPAD // NS
EPT = N_EDGES // NW
ECU = 176
NCHUNK = 56
TAIL = EPT - NCHUNK * ECU
NLPAD = 102400
LPT = NLPAD // NW
LC = 160
LCH = LPT // LC

_mesh = plsc.VectorSubcoreMesh(core_axis_name="c", subcore_axis_name="s")


def _make_agg(dim, with_cnt):
  out_type = [jax.ShapeDtypeStruct((NC * NPAD, dim), jnp.float32)]
  scratch = (
      [pltpu.VMEM_SHARED((NPAD, dim), jnp.float32)] +
      [pltpu.VMEM((ECU,), jnp.int32)] * 4 +
      [pltpu.VMEM((ECU,), jnp.int32)] * 4 +
      [pltpu.VMEM((ECU, dim), jnp.float32)] * 2 +
      [pltpu.VMEM((TAIL,), jnp.int32)] +
      [pltpu.SemaphoreType.DMA] * 12
  )
  if with_cnt:
    out_type.append(jax.ShapeDtypeStruct((NC * CPAD,), jnp.float32))
    scratch += [
        pltpu.VMEM_SHARED((CPAD,), jnp.float32),
        pltpu.VMEM((192,), jnp.float32),
        pltpu.VMEM((CRPT,), jnp.float32),
        pltpu.SemaphoreType.DMA,
        pltpu.SemaphoreType.DMA,
    ]

  nvec = dim // 16

  def body(feat, src_h, dst_h, *rest):
    if with_cnt:
      (part_o, cnt_o, acc_sh, s0, s1, s2, s3, d0, d1, d2, d3, rowsA, rowsB,
       tail_d, si0, si1, si2, si3, di0, di1, di2, di3, gsA, gsB, scA, scB,
       cnt_sh, ones_v, zrow_v, csA, csB) = rest
    else:
      (part_o, acc_sh, s0, s1, s2, s3, d0, d1, d2, d3, rowsA, rowsB,
       tail_d, si0, si1, si2, si3, di0, di1, di2, di3, gsA, gsB, scA, scB) = rest
      cnt_sh = ones_v = zrow_v = csA = csB = None
    src_v = (s0, s1, s2, s3)
    dst_v = (d0, d1, d2, d3)
    rows_v = (rowsA, rowsB)
    sisem = (si0, si1, si2, si3)
    disem = (di0, di1, di2, di3)
    gsem = (gsA, gsB)
    ssem = (scA, scB)
    csem = (csA, csB)

    c = lax.axis_index("c")
    s = lax.axis_index("s")
    wid = c * NS + s
    base_r = s * RPT
    ebase = wid * EPT

    def zrow(i, _):
      r = i // nvec
      k = i - r * nvec
      rowsA[r, pl.ds(k * 16, 16)] = jnp.zeros((16,), jnp.float32)
      return 0
    lax.fori_loop(0, ECU * nvec, zrow, 0)
    for j0 in range(0, RPT, ECU):
      n = min(ECU, RPT - j0)
      pltpu.sync_copy(rowsA.at[pl.ds(0, n)], acc_sh.at[pl.ds(base_r + j0, n)])

    if with_cnt:
      def fill1(i, _):
        ones_v[pl.ds(i * 16, 16)] = jnp.ones((16,), jnp.float32)
        return 0
      lax.fori_loop(0, 192 // 16, fill1, 0)
      def fill0(i, _):
        zrow_v[pl.ds(i * 16, 16)] = jnp.zeros((16,), jnp.float32)
        return 0
      lax.fori_loop(0, CRPT // 16, fill0, 0)
      pltpu.sync_copy(zrow_v, cnt_sh.at[pl.ds(s * CRPT, CRPT)])

    def issue_idx(cidx, q):
      off = ebase + cidx * ECU
      pltpu.async_copy(src_h.at[pl.ds(off, ECU)], src_v[q], sisem[q])
      pltpu.async_copy(dst_h.at[pl.ds(off, ECU)], dst_v[q], disem[q])

    def wait_idx(q):
      pltpu.make_async_copy(src_h.at[pl.ds(0, ECU)], src_v[q], sisem[q]).wait()
      pltpu.make_async_copy(dst_h.at[pl.ds(0, ECU)], dst_v[q], disem[q]).wait()

    def issue_gather(q, b):
      pltpu.async_copy(feat.at[src_v[q]], rows_v[b], gsem[b])

    def wait_gather(q, b):
      pltpu.make_async_copy(feat.at[src_v[q]], rows_v[b], gsem[b]).wait()

    def issue_scatter(q, b):
      pltpu.async_copy(rows_v[b], acc_sh.at[dst_v[q]], ssem[b], add=True)
      if with_cnt:
        pltpu.async_copy(ones_v.at[pl.ds(0, ECU)], cnt_sh.at[dst_v[q]],
                         csem[b], add=True)

    def wait_scatter(q, b):
      pltpu.make_async_copy(rows_v[b], acc_sh.at[dst_v[q]], ssem[b]).wait()
      if with_cnt:
        pltpu.make_async_copy(ones_v.at[pl.ds(0, ECU)], cnt_sh.at[dst_v[q]],
                              csem[b]).wait()

    issue_idx(0, 0)
    issue_idx(1, 1)
    plsc.subcore_barrier()
    wait_idx(0)
    issue_gather(0, 0)

    def quad(g, _):
      for c4 in range(4):
        b = c4 % 2
        o = 1 - b
        qn = (c4 + 1) % 4
        qp = (c4 + 2) % 4
        qo = (c4 + 3) % 4

        wait_gather(c4, b)

        def prefetch(gg=g, qq=qp, cc4=c4):
          issue_idx(4 * gg + cc4 + 2, qq)
        if c4 < 2:
          prefetch()
        else:
          pl.when(g < NCHUNK // 4 - 1)(prefetch)

        issue_scatter(c4, b)

        def drain(qq=qo, bb=o):
          wait_scatter(qq, bb)
        if c4 > 0:
          drain()
        else:
          pl.when(g > 0)(drain)

        def nxt(qq=qn, bb=o):
          wait_idx(qq)
          issue_gather(qq, bb)
        if c4 < 3:
          nxt()
        else:
          pl.when(g < NCHUNK // 4 - 1)(nxt)
      return 0
    lax.fori_loop(0, NCHUNK // 4, quad, 0)

    wait_scatter(3, 1)

    toff = ebase + NCHUNK * ECU
    pltpu.async_copy(src_h.at[pl.ds(toff, TAIL)], s0.at[pl.ds(0, TAIL)], si0)
    pltpu.async_copy(dst_h.at[pl.ds(toff, TAIL)], tail_d, di0)
    pltpu.make_async_copy(src_h.at[pl.ds(0, TAIL)], s0.at[pl.ds(0, TAIL)],
                          si0).wait()
    pltpu.make_async_copy(dst_h.at[pl.ds(0, TAIL)], tail_d, di0).wait()
    pltpu.async_copy(feat.at[s0.at[pl.ds(0, TAIL)]], rowsA.at[pl.ds(0, TAIL)],
                     gsA).wait()
    pltpu.async_copy(rowsA.at[pl.ds(0, TAIL)], acc_sh.at[tail_d], scA,
                     add=True).wait()
    if with_cnt:
      pltpu.async_copy(ones_v.at[pl.ds(0, TAIL)], cnt_sh.at[tail_d], csA,
                       add=True).wait()

    plsc.subcore_barrier()

    obase = c * NPAD + base_r
    pltpu.sync_copy(acc_sh.at[pl.ds(base_r, RPT)], part_o.at[pl.ds(obase, RPT)])
    if with_cnt:
      pltpu.sync_copy(cnt_sh.at[pl.ds(s * CRPT, CRPT)],
                      cnt_o.at[pl.ds(c * CPAD + s * CRPT, CRPT)])

  return pl.kernel(body, out_type=out_type, mesh=_mesh, scratch_types=scratch)


_agg_cnt = _make_agg(IN_DIM, True)
_agg_plain = _make_agg(HID_DIM, False)


def _tc_layer(part, cnt, feat, w_l, w_r, b, relu, hpad=None):
  n, din = feat.shape
  h = w_l.shape[0]
  hpad = h if hpad is None else hpad
  blk = 1000
  grid = (n // blk,)

  def body(part_ref, cnt_ref, feat_ref, wl_ref, wr_ref, b_ref, out_ref):
    p = part_ref[0] + part_ref[1]
    cn = jnp.maximum(cnt_ref[0] + cnt_ref[1], 1.0)
    agg = p / cn
    o = lax.dot_general(agg, wl_ref[...], (((1,), (1,)), ((), ())),
                        preferred_element_type=jnp.float32)
    o = o + lax.dot_general(feat_ref[...], wr_ref[...], (((1,), (1,)), ((), ())),
                            preferred_element_type=jnp.float32)
    o = o + b_ref[...][None, :]
    if relu:
      o = jnp.maximum(o, 0.0)
    if hpad > h:
      o = jnp.concatenate([o, jnp.zeros((o.shape[0], hpad - h), jnp.float32)],
                          axis=1)
    out_ref[...] = o

  return pl.pallas_call(
      body,
      grid=grid,
      in_specs=[
          pl.BlockSpec((2, blk, din), lambda i: (0, i, 0)),
          pl.BlockSpec((2, blk, 1), lambda i: (0, i, 0)),
          pl.BlockSpec((blk, din), lambda i: (i, 0)),
          pl.BlockSpec((h, din), lambda i: (0, 0)),
          pl.BlockSpec((h, din), lambda i: (0, 0)),
          pl.BlockSpec((h,), lambda i: (0,)),
      ],
      out_specs=pl.BlockSpec((blk, hpad), lambda i: (i, 0)),
      out_shape=jax.ShapeDtypeStruct((n, hpad), jnp.float32),
  )(part, cnt, feat, w_l, w_r, b)


def _make_decode(gdim, pdim):
  out_type = [jax.ShapeDtypeStruct((NLPAD, pdim), jnp.float32)]
  scratch = (
      [pltpu.VMEM((LC,), jnp.int32)] * 4 +
      [pltpu.VMEM((LC, gdim), jnp.float32)] * 4 +
      [pltpu.VMEM((LC, pdim), jnp.float32)] * 2 +
      [pltpu.SemaphoreType.DMA] * 10
  )

  def body(z_h, ls_h, ld_h, prod_o,
           siA, siB, diA, diB, sA, sB, dA, dB, pA, pB,
           gsA, gsB, gdA, gdB, isA, isB, idA, idB, wsA, wsB):
    si_v = (siA, siB)
    di_v = (diA, diB)
    srows = (sA, sB)
    drows = (dA, dB)
    prod_v = (pA, pB)
    gssem = (gsA, gsB)
    gdsem = (gdA, gdB)
    issem = (isA, isB)
    idsem = (idA, idB)
    wsem = (wsA, wsB)

    c = lax.axis_index("c")
    s = lax.axis_index("s")
    wid = c * NS + s
    tbase = wid * LPT

    def issue_idx(g, b):
      off = tbase + g * LC
      pltpu.async_copy(ls_h.at[pl.ds(off, LC)], si_v[b], issem[b])
      pltpu.async_copy(ld_h.at[pl.ds(off, LC)], di_v[b], idsem[b])

    def wait_idx(b):
      pltpu.make_async_copy(ls_h.at[pl.ds(0, LC)], si_v[b], issem[b]).wait()
      pltpu.make_async_copy(ld_h.at[pl.ds(0, LC)], di_v[b], idsem[b]).wait()

    def issue_gather(b):
      pltpu.async_copy(z_h.at[si_v[b]], srows[b], gssem[b])
      pltpu.async_copy(z_h.at[di_v[b]], drows[b], gdsem[b])

    def wait_gather(b):
      pltpu.make_async_copy(z_h.at[si_v[b]], srows[b], gssem[b]).wait()
      pltpu.make_async_copy(z_h.at[di_v[b]], drows[b], gdsem[b]).wait()

    def issue_write(g, b):
      pltpu.async_copy(prod_v[b], prod_o.at[pl.ds(tbase + g * LC, LC)], wsem[b])

    def wait_write(b):
      pltpu.make_async_copy(prod_v[b], prod_o.at[pl.ds(0, LC)], wsem[b]).wait()

    issue_idx(0, 0)
    wait_idx(0)
    issue_gather(0)

    def pair(g, _):
      for b in (0, 1):
        cidx = 2 * g + b
        o = 1 - b
        wait_gather(b)

        @pl.when(cidx + 1 < LCH)
        def _():
          issue_idx(cidx + 1, o)

        @pl.when(cidx > 0)
        def _():
          wait_write(o)

        @pl.when(cidx + 1 < LCH)
        def _():
          wait_idx(o)
          issue_gather(o)

        def prow(r, _):
          for k in range(pdim // 16):
            prod_v[b][r, pl.ds(k * 16, 16)] = (
                srows[b][r, pl.ds(k * 16, 16)] * drows[b][r, pl.ds(k * 16, 16)])
          return 0
        lax.fori_loop(0, LC, prow, 0)
        issue_write(cidx, b)
      return 0
    lax.fori_loop(0, LCH // 2, pair, 0)
    wait_write(1)
    plsc.subcore_barrier()

  return pl.kernel(
      body, out_type=out_type, mesh=_mesh, scratch_types=scratch,
      compiler_params=pltpu.CompilerParams(use_tc_tiling_on_sc=False))


_decode = _make_decode(OUT_DIM, OUT_DIM)


def _rowsum(prod):
  pairs = prod.reshape(NLPAD // 2, 128)
  nrow = N_LABEL // 2
  blk = 2000

  def body(p_ref, o_ref):
    p = p_ref[...]
    o_ref[...] = jnp.concatenate(
        [jnp.sum(p[:, :64], axis=1, keepdims=True),
         jnp.sum(p[:, 64:], axis=1, keepdims=True)], axis=1)

  out = pl.pallas_call(
      body,
      grid=(nrow // blk,),
      in_specs=[pl.BlockSpec((blk, 128), lambda i: (i, 0))],
      out_specs=pl.BlockSpec((blk, 2), lambda i: (i, 0)),
      out_shape=jax.ShapeDtypeStruct((nrow, 2), jnp.float32),
  )(pairs)
  return out.reshape(N_LABEL)


def kernel(x, edge_index, edge_label_index, W1_l, W1_r, b1, W2_l, W2_r, b2):
  x = x.astype(jnp.float32)
  src = jnp.asarray(edge_index[0], jnp.int32)
  dst = jnp.asarray(edge_index[1], jnp.int32)
  pad = jnp.zeros((NLPAD - N_LABEL,), jnp.int32)
  ls = jnp.concatenate([jnp.asarray(edge_label_index[0], jnp.int32), pad])
  ld = jnp.concatenate([jnp.asarray(edge_label_index[1], jnp.int32), pad])

  part1, cnt = _agg_cnt(x, src, dst)
  part1 = part1.reshape(NC, NPAD, IN_DIM)
  cnt2 = cnt.reshape(NC, CPAD, 1)
  h = _tc_layer(part1, cnt2, x, W1_l, W1_r, b1, relu=True)
  (part2,) = _agg_plain(h, src, dst)
  part2 = part2.reshape(NC, NPAD, HID_DIM)
  z = _tc_layer(part2, cnt2, h, W2_l, W2_r, b2, relu=False)
  (prod,) = _decode(z, ls, ld)
  return _rowsum(prod)


# speedup vs baseline: 7.8036x; 1.0003x over previous
import functools

import jax
import jax.numpy as jnp
from jax import lax
from jax.experimental import pallas as pl
from jax.experimental.pallas import tpu as pltpu
from jax.experimental.pallas import tpu_sc as plsc

N_NODES = 10000
IN_DIM = 128
HID_DIM = 128
OUT_DIM = 64
N_EDGES = 320000
N_LABEL = 100000

NC = 2
NS = 16
NW = NC * NS

NPAD = 10112
RPT = NPAD // NS
CPAD = 10240
CRPT = CPAD // NS
EPT = N_EDGES // NW
ECU = 176
NCHUNK = 56
TAIL = EPT - NCHUNK * ECU
NLPAD = 102400
LPT = NLPAD // NW
LC = 160
LCH = LPT // LC

_mesh = plsc.VectorSubcoreMesh(core_axis_name="c", subcore_axis_name="s")


def _make_agg(dim, with_cnt):
  out_type = [jax.ShapeDtypeStruct((NC * NPAD, dim), jnp.float32)]
  scratch = (
      [pltpu.VMEM_SHARED((NPAD, dim), jnp.float32)] +
      [pltpu.VMEM((ECU,), jnp.int32)] * 4 +
      [pltpu.VMEM((ECU,), jnp.int32)] * 4 +
      [pltpu.VMEM((ECU, dim), jnp.float32)] * 2 +
      [pltpu.VMEM((TAIL,), jnp.int32)] +
      [pltpu.SemaphoreType.DMA] * 12
  )
  if with_cnt:
    out_type.append(jax.ShapeDtypeStruct((NC * CPAD,), jnp.float32))
    scratch += [
        pltpu.VMEM_SHARED((CPAD,), jnp.float32),
        pltpu.VMEM((192,), jnp.float32),
        pltpu.VMEM((CRPT,), jnp.float32),
        pltpu.SemaphoreType.DMA,
        pltpu.SemaphoreType.DMA,
    ]

  nvec = dim // 16

  def body(feat, src_h, dst_h, *rest):
    if with_cnt:
      (part_o, cnt_o, acc_sh, s0, s1, s2, s3, d0, d1, d2, d3, rowsA, rowsB,
       tail_d, si0, si1, si2, si3, di0, di1, di2, di3, gsA, gsB, scA, scB,
       cnt_sh, ones_v, zrow_v, csA, csB) = rest
    else:
      (part_o, acc_sh, s0, s1, s2, s3, d0, d1, d2, d3, rowsA, rowsB,
       tail_d, si0, si1, si2, si3, di0, di1, di2, di3, gsA, gsB, scA, scB) = rest
      cnt_sh = ones_v = zrow_v = csA = csB = None
    src_v = (s0, s1, s2, s3)
    dst_v = (d0, d1, d2, d3)
    rows_v = (rowsA, rowsB)
    sisem = (si0, si1, si2, si3)
    disem = (di0, di1, di2, di3)
    gsem = (gsA, gsB)
    ssem = (scA, scB)
    csem = (csA, csB)

    c = lax.axis_index("c")
    s = lax.axis_index("s")
    wid = c * NS + s
    base_r = s * RPT
    ebase = wid * EPT

    def zrow(i, _):
      r = i // nvec
      k = i - r * nvec
      rowsA[r, pl.ds(k * 16, 16)] = jnp.zeros((16,), jnp.float32)
      return 0
    lax.fori_loop(0, ECU * nvec, zrow, 0)
    for j0 in range(0, RPT, ECU):
      n = min(ECU, RPT - j0)
      pltpu.sync_copy(rowsA.at[pl.ds(0, n)], acc_sh.at[pl.ds(base_r + j0, n)])

    if with_cnt:
      def fill1(i, _):
        ones_v[pl.ds(i * 16, 16)] = jnp.ones((16,), jnp.float32)
        return 0
      lax.fori_loop(0, 192 // 16, fill1, 0)
      def fill0(i, _):
        zrow_v[pl.ds(i * 16, 16)] = jnp.zeros((16,), jnp.float32)
        return 0
      lax.fori_loop(0, CRPT // 16, fill0, 0)
      pltpu.sync_copy(zrow_v, cnt_sh.at[pl.ds(s * CRPT, CRPT)])

    def issue_idx(cidx, q):
      off = ebase + cidx * ECU
      pltpu.async_copy(src_h.at[pl.ds(off, ECU)], src_v[q], sisem[q])
      pltpu.async_copy(dst_h.at[pl.ds(off, ECU)], dst_v[q], disem[q])

    def wait_idx(q):
      pltpu.make_async_copy(src_h.at[pl.ds(0, ECU)], src_v[q], sisem[q]).wait()
      pltpu.make_async_copy(dst_h.at[pl.ds(0, ECU)], dst_v[q], disem[q]).wait()

    def issue_gather(q, b):
      pltpu.async_copy(feat.at[src_v[q]], rows_v[b], gsem[b])

    def wait_gather(q, b):
      pltpu.make_async_copy(feat.at[src_v[q]], rows_v[b], gsem[b]).wait()

    def issue_scatter(q, b):
      pltpu.async_copy(rows_v[b], acc_sh.at[dst_v[q]], ssem[b], add=True)
      if with_cnt:
        pltpu.async_copy(ones_v.at[pl.ds(0, ECU)], cnt_sh.at[dst_v[q]],
                         csem[b], add=True)

    def wait_scatter(q, b):
      pltpu.make_async_copy(rows_v[b], acc_sh.at[dst_v[q]], ssem[b]).wait()
      if with_cnt:
        pltpu.make_async_copy(ones_v.at[pl.ds(0, ECU)], cnt_sh.at[dst_v[q]],
                              csem[b]).wait()

    issue_idx(0, 0)
    issue_idx(1, 1)
    plsc.subcore_barrier()
    wait_idx(0)
    issue_gather(0, 0)

    def quad(g, _):
      for c4 in range(4):
        b = c4 % 2
        o = 1 - b
        qn = (c4 + 1) % 4
        qp = (c4 + 2) % 4
        qo = (c4 + 3) % 4

        wait_gather(c4, b)

        def prefetch(gg=g, qq=qp, cc4=c4):
          issue_idx(4 * gg + cc4 + 2, qq)
        if c4 < 2:
          prefetch()
        else:
          pl.when(g < NCHUNK // 4 - 1)(prefetch)

        issue_scatter(c4, b)

        def drain(qq=qo, bb=o):
          wait_scatter(qq, bb)
        if c4 > 0:
          drain()
        else:
          pl.when(g > 0)(drain)

        def nxt(qq=qn, bb=o):
          wait_idx(qq)
          issue_gather(qq, bb)
        if c4 < 3:
          nxt()
        else:
          pl.when(g < NCHUNK // 4 - 1)(nxt)
      return 0
    lax.fori_loop(0, NCHUNK // 4, quad, 0)

    wait_scatter(3, 1)

    toff = ebase + NCHUNK * ECU
    pltpu.async_copy(src_h.at[pl.ds(toff, TAIL)], s0.at[pl.ds(0, TAIL)], si0)
    pltpu.async_copy(dst_h.at[pl.ds(toff, TAIL)], tail_d, di0)
    pltpu.make_async_copy(src_h.at[pl.ds(0, TAIL)], s0.at[pl.ds(0, TAIL)],
                          si0).wait()
    pltpu.make_async_copy(dst_h.at[pl.ds(0, TAIL)], tail_d, di0).wait()
    pltpu.async_copy(feat.at[s0.at[pl.ds(0, TAIL)]], rowsA.at[pl.ds(0, TAIL)],
                     gsA).wait()
    pltpu.async_copy(rowsA.at[pl.ds(0, TAIL)], acc_sh.at[tail_d], scA,
                     add=True).wait()
    if with_cnt:
      pltpu.async_copy(ones_v.at[pl.ds(0, TAIL)], cnt_sh.at[tail_d], csA,
                       add=True).wait()

    plsc.subcore_barrier()

    obase = c * NPAD + base_r
    pltpu.sync_copy(acc_sh.at[pl.ds(base_r, RPT)], part_o.at[pl.ds(obase, RPT)])
    if with_cnt:
      pltpu.sync_copy(cnt_sh.at[pl.ds(s * CRPT, CRPT)],
                      cnt_o.at[pl.ds(c * CPAD + s * CRPT, CRPT)])

  return pl.kernel(
      body, out_type=out_type, mesh=_mesh, scratch_types=scratch,
      compiler_params=pltpu.CompilerParams(use_tc_tiling_on_sc=False))


_agg_cnt = _make_agg(IN_DIM, True)
_agg_plain = _make_agg(HID_DIM, False)


def _tc_layer(part, cnt, feat, w_l, w_r, b, relu, hpad=None):
  n, din = feat.shape
  h = w_l.shape[0]
  hpad = h if hpad is None else hpad
  blk = 1000
  grid = (n // blk,)

  def body(part_ref, cnt_ref, feat_ref, wl_ref, wr_ref, b_ref, out_ref):
    p = part_ref[0] + part_ref[1]
    cn = jnp.maximum(cnt_ref[0] + cnt_ref[1], 1.0)
    agg = p / cn
    o = lax.dot_general(agg, wl_ref[...], (((1,), (1,)), ((), ())),
                        preferred_element_type=jnp.float32)
    o = o + lax.dot_general(feat_ref[...], wr_ref[...], (((1,), (1,)), ((), ())),
                            preferred_element_type=jnp.float32)
    o = o + b_ref[...][None, :]
    if relu:
      o = jnp.maximum(o, 0.0)
    if hpad > h:
      o = jnp.concatenate([o, jnp.zeros((o.shape[0], hpad - h), jnp.float32)],
                          axis=1)
    out_ref[...] = o

  return pl.pallas_call(
      body,
      grid=grid,
      in_specs=[
          pl.BlockSpec((2, blk, din), lambda i: (0, i, 0)),
          pl.BlockSpec((2, blk, 1), lambda i: (0, i, 0)),
          pl.BlockSpec((blk, din), lambda i: (i, 0)),
          pl.BlockSpec((h, din), lambda i: (0, 0)),
          pl.BlockSpec((h, din), lambda i: (0, 0)),
          pl.BlockSpec((h,), lambda i: (0,)),
      ],
      out_specs=pl.BlockSpec((blk, hpad), lambda i: (i, 0)),
      out_shape=jax.ShapeDtypeStruct((n, hpad), jnp.float32),
  )(part, cnt, feat, w_l, w_r, b)


def _make_decode(gdim, pdim):
  out_type = [jax.ShapeDtypeStruct((NLPAD, pdim), jnp.float32)]
  scratch = (
      [pltpu.VMEM((LC,), jnp.int32)] * 4 +
      [pltpu.VMEM((LC, gdim), jnp.float32)] * 4 +
      [pltpu.VMEM((LC, pdim), jnp.float32)] * 2 +
      [pltpu.SemaphoreType.DMA] * 10
  )

  def body(z_h, ls_h, ld_h, prod_o,
           siA, siB, diA, diB, sA, sB, dA, dB, pA, pB,
           gsA, gsB, gdA, gdB, isA, isB, idA, idB, wsA, wsB):
    si_v = (siA, siB)
    di_v = (diA, diB)
    srows = (sA, sB)
    drows = (dA, dB)
    prod_v = (pA, pB)
    gssem = (gsA, gsB)
    gdsem = (gdA, gdB)
    issem = (isA, isB)
    idsem = (idA, idB)
    wsem = (wsA, wsB)

    c = lax.axis_index("c")
    s = lax.axis_index("s")
    wid = c * NS + s
    tbase = wid * LPT

    def issue_idx(g, b):
      off = tbase + g * LC
      pltpu.async_copy(ls_h.at[pl.ds(off, LC)], si_v[b], issem[b])
      pltpu.async_copy(ld_h.at[pl.ds(off, LC)], di_v[b], idsem[b])

    def wait_idx(b):
      pltpu.make_async_copy(ls_h.at[pl.ds(0, LC)], si_v[b], issem[b]).wait()
      pltpu.make_async_copy(ld_h.at[pl.ds(0, LC)], di_v[b], idsem[b]).wait()

    def issue_gather(b):
      pltpu.async_copy(z_h.at[si_v[b]], srows[b], gssem[b])
      pltpu.async_copy(z_h.at[di_v[b]], drows[b], gdsem[b])

    def wait_gather(b):
      pltpu.make_async_copy(z_h.at[si_v[b]], srows[b], gssem[b]).wait()
      pltpu.make_async_copy(z_h.at[di_v[b]], drows[b], gdsem[b]).wait()

    def issue_write(g, b):
      pltpu.async_copy(prod_v[b], prod_o.at[pl.ds(tbase + g * LC, LC)], wsem[b])

    def wait_write(b):
      pltpu.make_async_copy(prod_v[b], prod_o.at[pl.ds(0, LC)], wsem[b]).wait()

    issue_idx(0, 0)
    wait_idx(0)
    issue_gather(0)

    def pair(g, _):
      for b in (0, 1):
        cidx = 2 * g + b
        o = 1 - b
        wait_gather(b)

        @pl.when(cidx + 1 < LCH)
        def _():
          issue_idx(cidx + 1, o)

        @pl.when(cidx > 0)
        def _():
          wait_write(o)

        @pl.when(cidx + 1 < LCH)
        def _():
          wait_idx(o)
          issue_gather(o)

        def prow(r, _):
          for k in range(pdim // 16):
            prod_v[b][r, pl.ds(k * 16, 16)] = (
                srows[b][r, pl.ds(k * 16, 16)] * drows[b][r, pl.ds(k * 16, 16)])
          return 0
        lax.fori_loop(0, LC, prow, 0)
        issue_write(cidx, b)
      return 0
    lax.fori_loop(0, LCH // 2, pair, 0)
    wait_write(1)
    plsc.subcore_barrier()

  return pl.kernel(
      body, out_type=out_type, mesh=_mesh, scratch_types=scratch,
      compiler_params=pltpu.CompilerParams(use_tc_tiling_on_sc=False))


_decode = _make_decode(OUT_DIM, OUT_DIM)


def _rowsum(prod):
  pairs = prod.reshape(NLPAD // 2, 128)
  nrow = N_LABEL // 2
  blk = 2000

  def body(p_ref, o_ref):
    p = p_ref[...]
    o_ref[...] = jnp.concatenate(
        [jnp.sum(p[:, :64], axis=1, keepdims=True),
         jnp.sum(p[:, 64:], axis=1, keepdims=True)], axis=1)

  out = pl.pallas_call(
      body,
      grid=(nrow // blk,),
      in_specs=[pl.BlockSpec((blk, 128), lambda i: (i, 0))],
      out_specs=pl.BlockSpec((blk, 2), lambda i: (i, 0)),
      out_shape=jax.ShapeDtypeStruct((nrow, 2), jnp.float32),
  )(pairs)
  return out.reshape(N_LABEL)


def kernel(x, edge_index, edge_label_index, W1_l, W1_r, b1, W2_l, W2_r, b2):
  x = x.astype(jnp.float32)
  src = jnp.asarray(edge_index[0], jnp.int32)
  dst = jnp.asarray(edge_index[1], jnp.int32)
  pad = jnp.zeros((NLPAD - N_LABEL,), jnp.int32)
  ls = jnp.concatenate([jnp.asarray(edge_label_index[0], jnp.int32), pad])
  ld = jnp.concatenate([jnp.asarray(edge_label_index[1], jnp.int32), pad])

  part1, cnt = _agg_cnt(x, src, dst)
  part1 = part1.reshape(NC, NPAD, IN_DIM)
  cnt2 = cnt.reshape(NC, CPAD, 1)
  h = _tc_layer(part1, cnt2, x, W1_l, W1_r, b1, relu=True)
  (part2,) = _agg_plain(h, src, dst)
  part2 = part2.reshape(NC, NPAD, HID_DIM)
  z = _tc_layer(part2, cnt2, h, W2_l, W2_r, b2, relu=False)
  (prod,) = _decode(z, ls, ld)
  return _rowsum(prod)


# speedup vs baseline: 8.0838x; 1.0359x over previous
import functools

import jax
import jax.numpy as jnp
from jax import lax
from jax.experimental import pallas as pl
from jax.experimental.pallas import tpu as pltpu
from jax.experimental.pallas import tpu_sc as plsc

N_NODES = 10000
IN_DIM = 128
HID_DIM = 128
OUT_DIM = 64
N_EDGES = 320000
N_LABEL = 100000

NC = 2
NS = 16
NW = NC * NS

NPAD = 10112
RPT = NPAD // NS
CPAD = 10240
CRPT = CPAD // NS
EPT = N_EDGES // NW
ECU = 176
NCHUNK = 56
TAIL = EPT - NCHUNK * ECU
NLPAD = 102400
LPT = NLPAD // NW
LC = 320
LCH = LPT // LC

_mesh = plsc.VectorSubcoreMesh(core_axis_name="c", subcore_axis_name="s")


def _make_agg(dim, with_cnt):
  out_type = [jax.ShapeDtypeStruct((NC * NPAD, dim), jnp.float32)]
  scratch = (
      [pltpu.VMEM_SHARED((NPAD, dim), jnp.float32)] +
      [pltpu.VMEM((ECU,), jnp.int32)] * 4 +
      [pltpu.VMEM((ECU,), jnp.int32)] * 4 +
      [pltpu.VMEM((ECU, dim), jnp.float32)] * 2 +
      [pltpu.VMEM((TAIL,), jnp.int32)] +
      [pltpu.SemaphoreType.DMA] * 12
  )
  if with_cnt:
    out_type.append(jax.ShapeDtypeStruct((NC * CPAD,), jnp.float32))
    scratch += [
        pltpu.VMEM_SHARED((CPAD,), jnp.float32),
        pltpu.VMEM((192,), jnp.float32),
        pltpu.VMEM((CRPT,), jnp.float32),
        pltpu.SemaphoreType.DMA,
        pltpu.SemaphoreType.DMA,
    ]

  nvec = dim // 16

  def body(feat, src_h, dst_h, *rest):
    if with_cnt:
      (part_o, cnt_o, acc_sh, s0, s1, s2, s3, d0, d1, d2, d3, rowsA, rowsB,
       tail_d, si0, si1, si2, si3, di0, di1, di2, di3, gsA, gsB, scA, scB,
       cnt_sh, ones_v, zrow_v, csA, csB) = rest
    else:
      (part_o, acc_sh, s0, s1, s2, s3, d0, d1, d2, d3, rowsA, rowsB,
       tail_d, si0, si1, si2, si3, di0, di1, di2, di3, gsA, gsB, scA, scB) = rest
      cnt_sh = ones_v = zrow_v = csA = csB = None
    src_v = (s0, s1, s2, s3)
    dst_v = (d0, d1, d2, d3)
    rows_v = (rowsA, rowsB)
    sisem = (si0, si1, si2, si3)
    disem = (di0, di1, di2, di3)
    gsem = (gsA, gsB)
    ssem = (scA, scB)
    csem = (csA, csB)

    c = lax.axis_index("c")
    s = lax.axis_index("s")
    wid = c * NS + s
    base_r = s * RPT
    ebase = wid * EPT

    def zrow(i, _):
      r = i // nvec
      k = i - r * nvec
      rowsA[r, pl.ds(k * 16, 16)] = jnp.zeros((16,), jnp.float32)
      return 0
    lax.fori_loop(0, ECU * nvec, zrow, 0)
    for j0 in range(0, RPT, ECU):
      n = min(ECU, RPT - j0)
      pltpu.sync_copy(rowsA.at[pl.ds(0, n)], acc_sh.at[pl.ds(base_r + j0, n)])

    if with_cnt:
      def fill1(i, _):
        ones_v[pl.ds(i * 16, 16)] = jnp.ones((16,), jnp.float32)
        return 0
      lax.fori_loop(0, 192 // 16, fill1, 0)
      def fill0(i, _):
        zrow_v[pl.ds(i * 16, 16)] = jnp.zeros((16,), jnp.float32)
        return 0
      lax.fori_loop(0, CRPT // 16, fill0, 0)
      pltpu.sync_copy(zrow_v, cnt_sh.at[pl.ds(s * CRPT, CRPT)])

    def issue_idx(cidx, q):
      off = ebase + cidx * ECU
      pltpu.async_copy(src_h.at[pl.ds(off, ECU)], src_v[q], sisem[q])
      pltpu.async_copy(dst_h.at[pl.ds(off, ECU)], dst_v[q], disem[q])

    def wait_idx(q):
      pltpu.make_async_copy(src_h.at[pl.ds(0, ECU)], src_v[q], sisem[q]).wait()
      pltpu.make_async_copy(dst_h.at[pl.ds(0, ECU)], dst_v[q], disem[q]).wait()

    def issue_gather(q, b):
      pltpu.async_copy(feat.at[src_v[q]], rows_v[b], gsem[b])

    def wait_gather(q, b):
      pltpu.make_async_copy(feat.at[src_v[q]], rows_v[b], gsem[b]).wait()

    def issue_scatter(q, b):
      pltpu.async_copy(rows_v[b], acc_sh.at[dst_v[q]], ssem[b], add=True)
      if with_cnt:
        pltpu.async_copy(ones_v.at[pl.ds(0, ECU)], cnt_sh.at[dst_v[q]],
                         csem[b], add=True)

    def wait_scatter(q, b):
      pltpu.make_async_copy(rows_v[b], acc_sh.at[dst_v[q]], ssem[b]).wait()
      if with_cnt:
        pltpu.make_async_copy(ones_v.at[pl.ds(0, ECU)], cnt_sh.at[dst_v[q]],
                              csem[b]).wait()

    issue_idx(0, 0)
    issue_idx(1, 1)
    plsc.subcore_barrier()
    wait_idx(0)
    issue_gather(0, 0)

    def quad(g, _):
      for c4 in range(4):
        b = c4 % 2
        o = 1 - b
        qn = (c4 + 1) % 4
        qp = (c4 + 2) % 4
        qo = (c4 + 3) % 4

        wait_gather(c4, b)

        def prefetch(gg=g, qq=qp, cc4=c4):
          issue_idx(4 * gg + cc4 + 2, qq)
        if c4 < 2:
          prefetch()
        else:
          pl.when(g < NCHUNK // 4 - 1)(prefetch)

        issue_scatter(c4, b)

        def drain(qq=qo, bb=o):
          wait_scatter(qq, bb)
        if c4 > 0:
          drain()
        else:
          pl.when(g > 0)(drain)

        def nxt(qq=qn, bb=o):
          wait_idx(qq)
          issue_gather(qq, bb)
        if c4 < 3:
          nxt()
        else:
          pl.when(g < NCHUNK // 4 - 1)(nxt)
      return 0
    lax.fori_loop(0, NCHUNK // 4, quad, 0)

    wait_scatter(3, 1)

    toff = ebase + NCHUNK * ECU
    pltpu.async_copy(src_h.at[pl.ds(toff, TAIL)], s0.at[pl.ds(0, TAIL)], si0)
    pltpu.async_copy(dst_h.at[pl.ds(toff, TAIL)], tail_d, di0)
    pltpu.make_async_copy(src_h.at[pl.ds(0, TAIL)], s0.at[pl.ds(0, TAIL)],
                          si0).wait()
    pltpu.make_async_copy(dst_h.at[pl.ds(0, TAIL)], tail_d, di0).wait()
    pltpu.async_copy(feat.at[s0.at[pl.ds(0, TAIL)]], rowsA.at[pl.ds(0, TAIL)],
                     gsA).wait()
    pltpu.async_copy(rowsA.at[pl.ds(0, TAIL)], acc_sh.at[tail_d], scA,
                     add=True).wait()
    if with_cnt:
      pltpu.async_copy(ones_v.at[pl.ds(0, TAIL)], cnt_sh.at[tail_d], csA,
                       add=True).wait()

    plsc.subcore_barrier()

    obase = c * NPAD + base_r
    pltpu.sync_copy(acc_sh.at[pl.ds(base_r, RPT)], part_o.at[pl.ds(obase, RPT)])
    if with_cnt:
      pltpu.sync_copy(cnt_sh.at[pl.ds(s * CRPT, CRPT)],
                      cnt_o.at[pl.ds(c * CPAD + s * CRPT, CRPT)])

  return pl.kernel(
      body, out_type=out_type, mesh=_mesh, scratch_types=scratch,
      compiler_params=pltpu.CompilerParams(use_tc_tiling_on_sc=False))


_agg_cnt = _make_agg(IN_DIM, True)
_agg_plain = _make_agg(HID_DIM, False)


def _tc_layer(part, cnt, feat, w_l, w_r, b, relu, hpad=None):
  n, din = feat.shape
  h = w_l.shape[0]
  hpad = h if hpad is None else hpad
  blk = 1000
  grid = (n // blk,)

  def body(part_ref, cnt_ref, feat_ref, wl_ref, wr_ref, b_ref, out_ref):
    p = part_ref[0] + part_ref[1]
    cn = jnp.maximum(cnt_ref[0] + cnt_ref[1], 1.0)
    agg = p / cn
    o = lax.dot_general(agg, wl_ref[...], (((1,), (1,)), ((), ())),
                        preferred_element_type=jnp.float32)
    o = o + lax.dot_general(feat_ref[...], wr_ref[...], (((1,), (1,)), ((), ())),
                            preferred_element_type=jnp.float32)
    o = o + b_ref[...][None, :]
    if relu:
      o = jnp.maximum(o, 0.0)
    if hpad > h:
      o = jnp.concatenate([o, jnp.zeros((o.shape[0], hpad - h), jnp.float32)],
                          axis=1)
    out_ref[...] = o

  return pl.pallas_call(
      body,
      grid=grid,
      in_specs=[
          pl.BlockSpec((2, blk, din), lambda i: (0, i, 0)),
          pl.BlockSpec((2, blk, 1), lambda i: (0, i, 0)),
          pl.BlockSpec((blk, din), lambda i: (i, 0)),
          pl.BlockSpec((h, din), lambda i: (0, 0)),
          pl.BlockSpec((h, din), lambda i: (0, 0)),
          pl.BlockSpec((h,), lambda i: (0,)),
      ],
      out_specs=pl.BlockSpec((blk, hpad), lambda i: (i, 0)),
      out_shape=jax.ShapeDtypeStruct((n, hpad), jnp.float32),
  )(part, cnt, feat, w_l, w_r, b)


def _make_decode(dim):
  out_type = [jax.ShapeDtypeStruct((NLPAD,), jnp.float32)]
  scratch = (
      [pltpu.VMEM((LC,), jnp.int32)] * 4 +
      [pltpu.VMEM((LC, dim), jnp.float32)] * 4 +
      [pltpu.VMEM((LPT,), jnp.float32)] +
      [pltpu.SemaphoreType.DMA] * 9
  )

  def body(z_h, ls_h, ld_h, dots_o,
           siA, siB, diA, diB, sA, sB, dA, dB, out_v,
           gsA, gsB, gdA, gdB, isA, isB, idA, idB, wsem):
    si_v = (siA, siB)
    di_v = (diA, diB)
    srows = (sA, sB)
    drows = (dA, dB)
    gssem = (gsA, gsB)
    gdsem = (gdA, gdB)
    issem = (isA, isB)
    idsem = (idA, idB)

    c = lax.axis_index("c")
    s = lax.axis_index("s")
    wid = c * NS + s
    tbase = wid * LPT
    lanes = lax.iota(jnp.int32, 16)

    def issue_idx(g, b):
      off = tbase + g * LC
      pltpu.async_copy(ls_h.at[pl.ds(off, LC)], si_v[b], issem[b])
      pltpu.async_copy(ld_h.at[pl.ds(off, LC)], di_v[b], idsem[b])

    def wait_idx(b):
      pltpu.make_async_copy(ls_h.at[pl.ds(0, LC)], si_v[b], issem[b]).wait()
      pltpu.make_async_copy(ld_h.at[pl.ds(0, LC)], di_v[b], idsem[b]).wait()

    def issue_gather(b):
      pltpu.async_copy(z_h.at[si_v[b]], srows[b], gssem[b])
      pltpu.async_copy(z_h.at[di_v[b]], drows[b], gdsem[b])

    def wait_gather(b):
      pltpu.make_async_copy(z_h.at[si_v[b]], srows[b], gssem[b]).wait()
      pltpu.make_async_copy(z_h.at[di_v[b]], drows[b], gdsem[b]).wait()

    issue_idx(0, 0)
    wait_idx(0)
    issue_gather(0)

    def pair(g, _):
      for b in (0, 1):
        cidx = 2 * g + b
        o = 1 - b
        wait_gather(b)

        @pl.when(cidx + 1 < LCH)
        def _():
          issue_idx(cidx + 1, o)
          wait_idx(o)
          issue_gather(o)

        def grp(j, _):
          rows16 = j * 16 + lanes

          def dquad(t, acc):
            for u in range(4):
              col = jnp.full((16,), 4 * t + u, jnp.int32)
              sv = plsc.load_gather(srows[b], [rows16, col])
              dv = plsc.load_gather(drows[b], [rows16, col])
              acc = acc + sv * dv
            return acc
          acc = lax.fori_loop(0, dim // 4, dquad,
                              jnp.zeros((16,), jnp.float32))
          out_v[pl.ds(cidx * LC + j * 16, 16)] = acc
          return 0
        lax.fori_loop(0, LC // 16, grp, 0)
      return 0
    lax.fori_loop(0, LCH // 2, pair, 0)

    pltpu.async_copy(out_v, dots_o.at[pl.ds(tbase, LPT)], wsem).wait()

  return pl.kernel(
      body, out_type=out_type, mesh=_mesh, scratch_types=scratch,
      compiler_params=pltpu.CompilerParams(use_tc_tiling_on_sc=False,
                                           needs_layout_passes=False))


_decode = _make_decode(OUT_DIM)


def _rowsum(prod):
  pairs = prod.reshape(NLPAD // 2, 128)
  nrow = N_LABEL // 2
  blk = 2000

  def body(p_ref, o_ref):
    p = p_ref[...]
    o_ref[...] = jnp.concatenate(
        [jnp.sum(p[:, :64], axis=1, keepdims=True),
         jnp.sum(p[:, 64:], axis=1, keepdims=True)], axis=1)

  out = pl.pallas_call(
      body,
      grid=(nrow // blk,),
      in_specs=[pl.BlockSpec((blk, 128), lambda i: (i, 0))],
      out_specs=pl.BlockSpec((blk, 2), lambda i: (i, 0)),
      out_shape=jax.ShapeDtypeStruct((nrow, 2), jnp.float32),
  )(pairs)
  return out.reshape(N_LABEL)


def kernel(x, edge_index, edge_label_index, W1_l, W1_r, b1, W2_l, W2_r, b2):
  x = x.astype(jnp.float32)
  src = jnp.asarray(edge_index[0], jnp.int32)
  dst = jnp.asarray(edge_index[1], jnp.int32)
  pad = jnp.zeros((NLPAD - N_LABEL,), jnp.int32)
  ls = jnp.concatenate([jnp.asarray(edge_label_index[0], jnp.int32), pad])
  ld = jnp.concatenate([jnp.asarray(edge_label_index[1], jnp.int32), pad])

  part1, cnt = _agg_cnt(x, src, dst)
  part1 = part1.reshape(NC, NPAD, IN_DIM)
  cnt2 = cnt.reshape(NC, CPAD, 1)
  h = _tc_layer(part1, cnt2, x, W1_l, W1_r, b1, relu=True)
  (part2,) = _agg_plain(h, src, dst)
  part2 = part2.reshape(NC, NPAD, HID_DIM)
  z = _tc_layer(part2, cnt2, h, W2_l, W2_r, b2, relu=False)
  (dots,) = _decode(z, ls, ld)
  return dots[:N_LABEL]


# speedup vs baseline: 8.0854x; 1.0002x over previous
import functools

import jax
import jax.numpy as jnp
from jax import lax
from jax.experimental import pallas as pl
from jax.experimental.pallas import tpu as pltpu
from jax.experimental.pallas import tpu_sc as plsc

N_NODES = 10000
IN_DIM = 128
HID_DIM = 128
OUT_DIM = 64
N_EDGES = 320000
N_LABEL = 100000

NC = 2
NS = 16
NW = NC * NS

NPAD = 10112
RPT = NPAD // NS
CPAD = 10240
CRPT = CPAD // NS
EPT = N_EDGES // NW
ECU = 176
NCHUNK = 56
TAIL = EPT - NCHUNK * ECU
NLPAD = 102400
LPT = NLPAD // NW
LC = 320
LCH = LPT // LC

_mesh = plsc.VectorSubcoreMesh(core_axis_name="c", subcore_axis_name="s")


def _make_agg(dim, with_cnt):
  out_type = [jax.ShapeDtypeStruct((NC, NPAD, dim), jnp.float32)]
  scratch = (
      [pltpu.VMEM_SHARED((NPAD, dim), jnp.float32)] +
      [pltpu.VMEM((ECU,), jnp.int32)] * 4 +
      [pltpu.VMEM((ECU,), jnp.int32)] * 4 +
      [pltpu.VMEM((ECU, dim), jnp.float32)] * 2 +
      [pltpu.VMEM((TAIL,), jnp.int32)] +
      [pltpu.SemaphoreType.DMA] * 12
  )
  if with_cnt:
    out_type.append(jax.ShapeDtypeStruct((NC, CPAD), jnp.float32))
    scratch += [
        pltpu.VMEM_SHARED((CPAD,), jnp.float32),
        pltpu.VMEM((192,), jnp.float32),
        pltpu.VMEM((CRPT,), jnp.float32),
        pltpu.SemaphoreType.DMA,
        pltpu.SemaphoreType.DMA,
    ]

  nvec = dim // 16

  def body(feat, src_h, dst_h, *rest):
    if with_cnt:
      (part_o, cnt_o, acc_sh, s0, s1, s2, s3, d0, d1, d2, d3, rowsA, rowsB,
       tail_d, si0, si1, si2, si3, di0, di1, di2, di3, gsA, gsB, scA, scB,
       cnt_sh, ones_v, zrow_v, csA, csB) = rest
    else:
      (part_o, acc_sh, s0, s1, s2, s3, d0, d1, d2, d3, rowsA, rowsB,
       tail_d, si0, si1, si2, si3, di0, di1, di2, di3, gsA, gsB, scA, scB) = rest
      cnt_sh = ones_v = zrow_v = csA = csB = None
    src_v = (s0, s1, s2, s3)
    dst_v = (d0, d1, d2, d3)
    rows_v = (rowsA, rowsB)
    sisem = (si0, si1, si2, si3)
    disem = (di0, di1, di2, di3)
    gsem = (gsA, gsB)
    ssem = (scA, scB)
    csem = (csA, csB)

    c = lax.axis_index("c")
    s = lax.axis_index("s")
    wid = c * NS + s
    base_r = s * RPT
    ebase = wid * EPT

    def zrow(i, _):
      r = i // nvec
      k = i - r * nvec
      rowsA[r, pl.ds(k * 16, 16)] = jnp.zeros((16,), jnp.float32)
      return 0
    lax.fori_loop(0, ECU * nvec, zrow, 0)
    for j0 in range(0, RPT, ECU):
      n = min(ECU, RPT - j0)
      pltpu.sync_copy(rowsA.at[pl.ds(0, n)], acc_sh.at[pl.ds(base_r + j0, n)])

    if with_cnt:
      def fill1(i, _):
        ones_v[pl.ds(i * 16, 16)] = jnp.ones((16,), jnp.float32)
        return 0
      lax.fori_loop(0, 192 // 16, fill1, 0)
      def fill0(i, _):
        zrow_v[pl.ds(i * 16, 16)] = jnp.zeros((16,), jnp.float32)
        return 0
      lax.fori_loop(0, CRPT // 16, fill0, 0)
      pltpu.sync_copy(zrow_v, cnt_sh.at[pl.ds(s * CRPT, CRPT)])

    def issue_idx(cidx, q):
      off = ebase + cidx * ECU
      pltpu.async_copy(src_h.at[pl.ds(off, ECU)], src_v[q], sisem[q])
      pltpu.async_copy(dst_h.at[pl.ds(off, ECU)], dst_v[q], disem[q])

    def wait_idx(q):
      pltpu.make_async_copy(src_h.at[pl.ds(0, ECU)], src_v[q], sisem[q]).wait()
      pltpu.make_async_copy(dst_h.at[pl.ds(0, ECU)], dst_v[q], disem[q]).wait()

    def issue_gather(q, b):
      pltpu.async_copy(feat.at[src_v[q]], rows_v[b], gsem[b])

    def wait_gather(q, b):
      pltpu.make_async_copy(feat.at[src_v[q]], rows_v[b], gsem[b]).wait()

    def issue_scatter(q, b):
      pltpu.async_copy(rows_v[b], acc_sh.at[dst_v[q]], ssem[b], add=True)
      if with_cnt:
        pltpu.async_copy(ones_v.at[pl.ds(0, ECU)], cnt_sh.at[dst_v[q]],
                         csem[b], add=True)

    def wait_scatter(q, b):
      pltpu.make_async_copy(rows_v[b], acc_sh.at[dst_v[q]], ssem[b]).wait()
      if with_cnt:
        pltpu.make_async_copy(ones_v.at[pl.ds(0, ECU)], cnt_sh.at[dst_v[q]],
                              csem[b]).wait()

    issue_idx(0, 0)
    issue_idx(1, 1)
    plsc.subcore_barrier()
    wait_idx(0)
    issue_gather(0, 0)

    def quad(g, _):
      for c4 in range(4):
        b = c4 % 2
        o = 1 - b
        qn = (c4 + 1) % 4
        qp = (c4 + 2) % 4
        qo = (c4 + 3) % 4

        wait_gather(c4, b)

        def prefetch(gg=g, qq=qp, cc4=c4):
          issue_idx(4 * gg + cc4 + 2, qq)
        if c4 < 2:
          prefetch()
        else:
          pl.when(g < NCHUNK // 4 - 1)(prefetch)

        issue_scatter(c4, b)

        def drain(qq=qo, bb=o):
          wait_scatter(qq, bb)
        if c4 > 0:
          drain()
        else:
          pl.when(g > 0)(drain)

        def nxt(qq=qn, bb=o):
          wait_idx(qq)
          issue_gather(qq, bb)
        if c4 < 3:
          nxt()
        else:
          pl.when(g < NCHUNK // 4 - 1)(nxt)
      return 0
    lax.fori_loop(0, NCHUNK // 4, quad, 0)

    wait_scatter(3, 1)

    toff = ebase + NCHUNK * ECU
    pltpu.async_copy(src_h.at[pl.ds(toff, TAIL)], s0.at[pl.ds(0, TAIL)], si0)
    pltpu.async_copy(dst_h.at[pl.ds(toff, TAIL)], tail_d, di0)
    pltpu.make_async_copy(src_h.at[pl.ds(0, TAIL)], s0.at[pl.ds(0, TAIL)],
                          si0).wait()
    pltpu.make_async_copy(dst_h.at[pl.ds(0, TAIL)], tail_d, di0).wait()
    pltpu.async_copy(feat.at[s0.at[pl.ds(0, TAIL)]], rowsA.at[pl.ds(0, TAIL)],
                     gsA).wait()
    pltpu.async_copy(rowsA.at[pl.ds(0, TAIL)], acc_sh.at[tail_d], scA,
                     add=True).wait()
    if with_cnt:
      pltpu.async_copy(ones_v.at[pl.ds(0, TAIL)], cnt_sh.at[tail_d], csA,
                       add=True).wait()

    plsc.subcore_barrier()

    pltpu.sync_copy(acc_sh.at[pl.ds(base_r, RPT)],
                    part_o.at[c, pl.ds(base_r, RPT)])
    if with_cnt:
      pltpu.sync_copy(cnt_sh.at[pl.ds(s * CRPT, CRPT)],
                      cnt_o.at[c, pl.ds(s * CRPT, CRPT)])

  return pl.kernel(
      body, out_type=out_type, mesh=_mesh, scratch_types=scratch,
      compiler_params=pltpu.CompilerParams(use_tc_tiling_on_sc=False))


_agg_cnt = _make_agg(IN_DIM, True)
_agg_plain = _make_agg(HID_DIM, False)


def _tc_layer(part, cnt, feat, w_l, w_r, b, relu, hpad=None):
  n, din = feat.shape
  h = w_l.shape[0]
  hpad = h if hpad is None else hpad
  blk = 1000
  grid = (n // blk,)

  def body(part_ref, cnt_ref, feat_ref, wl_ref, wr_ref, b_ref, out_ref):
    p = part_ref[0] + part_ref[1]
    cn = jnp.maximum(cnt_ref[0] + cnt_ref[1], 1.0)
    agg = p / cn
    o = lax.dot_general(agg, wl_ref[...], (((1,), (1,)), ((), ())),
                        preferred_element_type=jnp.float32)
    o = o + lax.dot_general(feat_ref[...], wr_ref[...], (((1,), (1,)), ((), ())),
                            preferred_element_type=jnp.float32)
    o = o + b_ref[...][None, :]
    if relu:
      o = jnp.maximum(o, 0.0)
    if hpad > h:
      o = jnp.concatenate([o, jnp.zeros((o.shape[0], hpad - h), jnp.float32)],
                          axis=1)
    out_ref[...] = o

  return pl.pallas_call(
      body,
      grid=grid,
      in_specs=[
          pl.BlockSpec((2, blk, din), lambda i: (0, i, 0)),
          pl.BlockSpec((2, blk, 1), lambda i: (0, i, 0)),
          pl.BlockSpec((blk, din), lambda i: (i, 0)),
          pl.BlockSpec((h, din), lambda i: (0, 0)),
          pl.BlockSpec((h, din), lambda i: (0, 0)),
          pl.BlockSpec((h,), lambda i: (0,)),
      ],
      out_specs=pl.BlockSpec((blk, hpad), lambda i: (i, 0)),
      out_shape=jax.ShapeDtypeStruct((n, hpad), jnp.float32),
  )(part, cnt, feat, w_l, w_r, b)


def _make_decode(dim):
  out_type = [jax.ShapeDtypeStruct((NLPAD,), jnp.float32)]
  scratch = (
      [pltpu.VMEM((LC,), jnp.int32)] * 4 +
      [pltpu.VMEM((LC, dim), jnp.float32)] * 4 +
      [pltpu.VMEM((LPT,), jnp.float32)] +
      [pltpu.SemaphoreType.DMA] * 9
  )

  def body(z_h, ls_h, ld_h, dots_o,
           siA, siB, diA, diB, sA, sB, dA, dB, out_v,
           gsA, gsB, gdA, gdB, isA, isB, idA, idB, wsem):
    si_v = (siA, siB)
    di_v = (diA, diB)
    srows = (sA, sB)
    drows = (dA, dB)
    gssem = (gsA, gsB)
    gdsem = (gdA, gdB)
    issem = (isA, isB)
    idsem = (idA, idB)

    c = lax.axis_index("c")
    s = lax.axis_index("s")
    wid = c * NS + s
    tbase = wid * LPT
    lanes = lax.iota(jnp.int32, 16)

    def issue_idx(g, b):
      off = tbase + g * LC
      pltpu.async_copy(ls_h.at[pl.ds(off, LC)], si_v[b], issem[b])
      pltpu.async_copy(ld_h.at[pl.ds(off, LC)], di_v[b], idsem[b])

    def wait_idx(b):
      pltpu.make_async_copy(ls_h.at[pl.ds(0, LC)], si_v[b], issem[b]).wait()
      pltpu.make_async_copy(ld_h.at[pl.ds(0, LC)], di_v[b], idsem[b]).wait()

    def issue_gather(b):
      pltpu.async_copy(z_h.at[si_v[b]], srows[b], gssem[b])
      pltpu.async_copy(z_h.at[di_v[b]], drows[b], gdsem[b])

    def wait_gather(b):
      pltpu.make_async_copy(z_h.at[si_v[b]], srows[b], gssem[b]).wait()
      pltpu.make_async_copy(z_h.at[di_v[b]], drows[b], gdsem[b]).wait()

    issue_idx(0, 0)
    wait_idx(0)
    issue_gather(0)

    def pair(g, _):
      for b in (0, 1):
        cidx = 2 * g + b
        o = 1 - b
        wait_gather(b)

        @pl.when(cidx + 1 < LCH)
        def _():
          issue_idx(cidx + 1, o)
          wait_idx(o)
          issue_gather(o)

        def grp(j, _):
          rows16 = j * 16 + lanes

          def dquad(t, acc):
            for u in range(4):
              col = jnp.full((16,), 4 * t + u, jnp.int32)
              sv = plsc.load_gather(srows[b], [rows16, col])
              dv = plsc.load_gather(drows[b], [rows16, col])
              acc = acc + sv * dv
            return acc
          acc = lax.fori_loop(0, dim // 4, dquad,
                              jnp.zeros((16,), jnp.float32))
          out_v[pl.ds(cidx * LC + j * 16, 16)] = acc
          return 0
        lax.fori_loop(0, LC // 16, grp, 0)
      return 0
    lax.fori_loop(0, LCH // 2, pair, 0)

    pltpu.async_copy(out_v, dots_o.at[pl.ds(tbase, LPT)], wsem).wait()

  return pl.kernel(
      body, out_type=out_type, mesh=_mesh, scratch_types=scratch,
      compiler_params=pltpu.CompilerParams(use_tc_tiling_on_sc=False,
                                           needs_layout_passes=False))


_decode = _make_decode(OUT_DIM)


def kernel(x, edge_index, edge_label_index, W1_l, W1_r, b1, W2_l, W2_r, b2):
  x = x.astype(jnp.float32)
  src = jnp.asarray(edge_index[0], jnp.int32)
  dst = jnp.asarray(edge_index[1], jnp.int32)
  pad = jnp.zeros((NLPAD - N_LABEL,), jnp.int32)
  ls = jnp.concatenate([jnp.asarray(edge_label_index[0], jnp.int32), pad])
  ld = jnp.concatenate([jnp.asarray(edge_label_index[1], jnp.int32), pad])

  part1, cnt = _agg_cnt(x, src, dst)
  cnt2 = cnt.reshape(NC, CPAD, 1)
  h = _tc_layer(part1, cnt2, x, W1_l, W1_r, b1, relu=True)
  (part2,) = _agg_plain(h, src, dst)
  z = _tc_layer(part2, cnt2, h, W2_l, W2_r, b2, relu=False)
  (dots,) = _decode(z, ls, ld)
  return dots[:N_LABEL]


# speedup vs baseline: 8.0890x; 1.0004x over previous
import jax
import jax.numpy as jnp
from jax import lax
from jax.experimental import pallas as pl
from jax.experimental.pallas import tpu as pltpu
from jax.experimental.pallas import tpu_sc as plsc

N_NODES = 10000
IN_DIM = 128
HID_DIM = 128
OUT_DIM = 64
N_EDGES = 320000
N_LABEL = 100000

NC = 2
NS = 16
NW = NC * NS

NPAD = 10112
RPT = NPAD // NS
CPAD = 10240
CRPT = CPAD // NS
EPT = N_EDGES // NW
ECU = 176
NCHUNK = 56
TAIL = EPT - NCHUNK * ECU
NLPAD = 102400
LPT = NLPAD // NW
LC = 320
LCH = LPT // LC

_mesh = plsc.VectorSubcoreMesh(core_axis_name="c", subcore_axis_name="s")


def _make_agg(dim, with_cnt):
  out_type = [jax.ShapeDtypeStruct((NC, NPAD, dim), jnp.float32)]
  scratch = (
      [pltpu.VMEM_SHARED((NPAD, dim), jnp.float32)] +
      [pltpu.VMEM((ECU,), jnp.int32)] * 4 +
      [pltpu.VMEM((ECU,), jnp.int32)] * 4 +
      [pltpu.VMEM((ECU, dim), jnp.float32)] * 2 +
      [pltpu.VMEM((TAIL,), jnp.int32)] +
      [pltpu.SemaphoreType.DMA] * 12
  )
  if with_cnt:
    out_type.append(jax.ShapeDtypeStruct((NC, CPAD), jnp.float32))
    scratch += [
        pltpu.VMEM_SHARED((CPAD,), jnp.float32),
        pltpu.VMEM((192,), jnp.float32),
        pltpu.VMEM((CRPT,), jnp.float32),
        pltpu.SemaphoreType.DMA,
        pltpu.SemaphoreType.DMA,
    ]

  nvec = dim // 16

  def body(feat, src_h, dst_h, *rest):
    if with_cnt:
      (part_o, cnt_o, acc_sh, s0, s1, s2, s3, d0, d1, d2, d3, rowsA, rowsB,
       tail_d, si0, si1, si2, si3, di0, di1, di2, di3, gsA, gsB, scA, scB,
       cnt_sh, ones_v, zrow_v, csA, csB) = rest
    else:
      (part_o, acc_sh, s0, s1, s2, s3, d0, d1, d2, d3, rowsA, rowsB,
       tail_d, si0, si1, si2, si3, di0, di1, di2, di3, gsA, gsB, scA, scB) = rest
      cnt_sh = ones_v = zrow_v = csA = csB = None
    src_v = (s0, s1, s2, s3)
    dst_v = (d0, d1, d2, d3)
    rows_v = (rowsA, rowsB)
    sisem = (si0, si1, si2, si3)
    disem = (di0, di1, di2, di3)
    gsem = (gsA, gsB)
    ssem = (scA, scB)
    csem = (csA, csB)

    c = lax.axis_index("c")
    s = lax.axis_index("s")
    wid = c * NS + s
    base_r = s * RPT
    ebase = wid * EPT

    def zrow(i, _):
      r = i // nvec
      k = i - r * nvec
      rowsA[r, pl.ds(k * 16, 16)] = jnp.zeros((16,), jnp.float32)
      return 0
    lax.fori_loop(0, ECU * nvec, zrow, 0)
    for j0 in range(0, RPT, ECU):
      n = min(ECU, RPT - j0)
      pltpu.sync_copy(rowsA.at[pl.ds(0, n)], acc_sh.at[pl.ds(base_r + j0, n)])

    if with_cnt:
      def fill1(i, _):
        ones_v[pl.ds(i * 16, 16)] = jnp.ones((16,), jnp.float32)
        return 0
      lax.fori_loop(0, 192 // 16, fill1, 0)
      def fill0(i, _):
        zrow_v[pl.ds(i * 16, 16)] = jnp.zeros((16,), jnp.float32)
        return 0
      lax.fori_loop(0, CRPT // 16, fill0, 0)
      pltpu.sync_copy(zrow_v, cnt_sh.at[pl.ds(s * CRPT, CRPT)])

    def issue_idx(cidx, q):
      off = ebase + cidx * ECU
      pltpu.async_copy(src_h.at[pl.ds(off, ECU)], src_v[q], sisem[q])
      pltpu.async_copy(dst_h.at[pl.ds(off, ECU)], dst_v[q], disem[q])

    def wait_idx(q):
      pltpu.make_async_copy(src_h.at[pl.ds(0, ECU)], src_v[q], sisem[q]).wait()
      pltpu.make_async_copy(dst_h.at[pl.ds(0, ECU)], dst_v[q], disem[q]).wait()

    def issue_gather(q, b):
      pltpu.async_copy(feat.at[src_v[q]], rows_v[b], gsem[b])

    def wait_gather(q, b):
      pltpu.make_async_copy(feat.at[src_v[q]], rows_v[b], gsem[b]).wait()

    def issue_scatter(q, b):
      pltpu.async_copy(rows_v[b], acc_sh.at[dst_v[q]], ssem[b], add=True)
      if with_cnt:
        pltpu.async_copy(ones_v.at[pl.ds(0, ECU)], cnt_sh.at[dst_v[q]],
                         csem[b], add=True)

    def wait_scatter(q, b):
      pltpu.make_async_copy(rows_v[b], acc_sh.at[dst_v[q]], ssem[b]).wait()
      if with_cnt:
        pltpu.make_async_copy(ones_v.at[pl.ds(0, ECU)], cnt_sh.at[dst_v[q]],
                              csem[b]).wait()

    issue_idx(0, 0)
    issue_idx(1, 1)
    plsc.subcore_barrier()
    wait_idx(0)
    issue_gather(0, 0)

    def quad(g, _):
      for c4 in range(4):
        b = c4 % 2
        o = 1 - b
        qn = (c4 + 1) % 4
        qp = (c4 + 2) % 4
        qo = (c4 + 3) % 4

        wait_gather(c4, b)

        def prefetch(gg=g, qq=qp, cc4=c4):
          issue_idx(4 * gg + cc4 + 2, qq)
        if c4 < 2:
          prefetch()
        else:
          pl.when(g < NCHUNK // 4 - 1)(prefetch)

        issue_scatter(c4, b)

        def drain(qq=qo, bb=o):
          wait_scatter(qq, bb)
        if c4 > 0:
          drain()
        else:
          pl.when(g > 0)(drain)

        def nxt(qq=qn, bb=o):
          wait_idx(qq)
          issue_gather(qq, bb)
        if c4 < 3:
          nxt()
        else:
          pl.when(g < NCHUNK // 4 - 1)(nxt)
      return 0
    lax.fori_loop(0, NCHUNK // 4, quad, 0)

    wait_scatter(3, 1)

    toff = ebase + NCHUNK * ECU
    pltpu.async_copy(src_h.at[pl.ds(toff, TAIL)], s0.at[pl.ds(0, TAIL)], si0)
    pltpu.async_copy(dst_h.at[pl.ds(toff, TAIL)], tail_d, di0)
    pltpu.make_async_copy(src_h.at[pl.ds(0, TAIL)], s0.at[pl.ds(0, TAIL)],
                          si0).wait()
    pltpu.make_async_copy(dst_h.at[pl.ds(0, TAIL)], tail_d, di0).wait()
    pltpu.async_copy(feat.at[s0.at[pl.ds(0, TAIL)]], rowsA.at[pl.ds(0, TAIL)],
                     gsA).wait()
    pltpu.async_copy(rowsA.at[pl.ds(0, TAIL)], acc_sh.at[tail_d], scA,
                     add=True).wait()
    if with_cnt:
      pltpu.async_copy(ones_v.at[pl.ds(0, TAIL)], cnt_sh.at[tail_d], csA,
                       add=True).wait()

    plsc.subcore_barrier()

    pltpu.sync_copy(acc_sh.at[pl.ds(base_r, RPT)],
                    part_o.at[c, pl.ds(base_r, RPT)])
    if with_cnt:
      pltpu.sync_copy(cnt_sh.at[pl.ds(s * CRPT, CRPT)],
                      cnt_o.at[c, pl.ds(s * CRPT, CRPT)])

  return pl.kernel(
      body, out_type=out_type, mesh=_mesh, scratch_types=scratch,
      compiler_params=pltpu.CompilerParams(use_tc_tiling_on_sc=False))


_agg_cnt = _make_agg(IN_DIM, True)
_agg_plain = _make_agg(HID_DIM, False)


def _tc_layer(part, cnt, feat, w_l, w_r, b, relu, hpad=None):
  n, din = feat.shape
  h = w_l.shape[0]
  hpad = h if hpad is None else hpad
  blk = 1000
  grid = (n // blk,)

  def body(part_ref, cnt_ref, feat_ref, wl_ref, wr_ref, b_ref, out_ref):
    p = part_ref[0] + part_ref[1]
    cn = jnp.maximum(cnt_ref[0] + cnt_ref[1], 1.0)
    agg = p / cn
    o = lax.dot_general(agg, wl_ref[...], (((1,), (1,)), ((), ())),
                        preferred_element_type=jnp.float32)
    o = o + lax.dot_general(feat_ref[...], wr_ref[...], (((1,), (1,)), ((), ())),
                            preferred_element_type=jnp.float32)
    o = o + b_ref[...][None, :]
    if relu:
      o = jnp.maximum(o, 0.0)
    if hpad > h:
      o = jnp.concatenate([o, jnp.zeros((o.shape[0], hpad - h), jnp.float32)],
                          axis=1)
    out_ref[...] = o

  return pl.pallas_call(
      body,
      grid=grid,
      in_specs=[
          pl.BlockSpec((2, blk, din), lambda i: (0, i, 0)),
          pl.BlockSpec((2, blk, 1), lambda i: (0, i, 0)),
          pl.BlockSpec((blk, din), lambda i: (i, 0)),
          pl.BlockSpec((h, din), lambda i: (0, 0)),
          pl.BlockSpec((h, din), lambda i: (0, 0)),
          pl.BlockSpec((h,), lambda i: (0,)),
      ],
      out_specs=pl.BlockSpec((blk, hpad), lambda i: (i, 0)),
      out_shape=jax.ShapeDtypeStruct((n, hpad), jnp.float32),
  )(part, cnt, feat, w_l, w_r, b)


def _make_decode(dim):
  out_type = [jax.ShapeDtypeStruct((NLPAD,), jnp.float32)]
  scratch = (
      [pltpu.VMEM((LC,), jnp.int32)] * 4 +
      [pltpu.VMEM((LC, dim), jnp.float32)] * 4 +
      [pltpu.VMEM((LPT,), jnp.float32)] +
      [pltpu.SemaphoreType.DMA] * 9
  )

  def body(z_h, ls_h, ld_h, dots_o,
           siA, siB, diA, diB, sA, sB, dA, dB, out_v,
           gsA, gsB, gdA, gdB, isA, isB, idA, idB, wsem):
    si_v = (siA, siB)
    di_v = (diA, diB)
    srows = (sA, sB)
    drows = (dA, dB)
    gssem = (gsA, gsB)
    gdsem = (gdA, gdB)
    issem = (isA, isB)
    idsem = (idA, idB)

    c = lax.axis_index("c")
    s = lax.axis_index("s")
    wid = c * NS + s
    tbase = wid * LPT
    lanes = lax.iota(jnp.int32, 16)

    def issue_idx(g, b):
      off = tbase + g * LC
      pltpu.async_copy(ls_h.at[pl.ds(off, LC)], si_v[b], issem[b])
      pltpu.async_copy(ld_h.at[pl.ds(off, LC)], di_v[b], idsem[b])

    def wait_idx(b):
      pltpu.make_async_copy(ls_h.at[pl.ds(0, LC)], si_v[b], issem[b]).wait()
      pltpu.make_async_copy(ld_h.at[pl.ds(0, LC)], di_v[b], idsem[b]).wait()

    def issue_gather(b):
      pltpu.async_copy(z_h.at[si_v[b]], srows[b], gssem[b])
      pltpu.async_copy(z_h.at[di_v[b]], drows[b], gdsem[b])

    def wait_gather(b):
      pltpu.make_async_copy(z_h.at[si_v[b]], srows[b], gssem[b]).wait()
      pltpu.make_async_copy(z_h.at[di_v[b]], drows[b], gdsem[b]).wait()

    issue_idx(0, 0)
    wait_idx(0)
    issue_gather(0)

    def pair(g, _):
      for b in (0, 1):
        cidx = 2 * g + b
        o = 1 - b
        wait_gather(b)

        @pl.when(cidx + 1 < LCH)
        def _():
          issue_idx(cidx + 1, o)
          wait_idx(o)
          issue_gather(o)

        def grp(j, _):
          rows16 = j * 16 + lanes

          def dquad(t, acc):
            for u in range(4):
              col = jnp.full((16,), 4 * t + u, jnp.int32)
              sv = plsc.load_gather(srows[b], [rows16, col])
              dv = plsc.load_gather(drows[b], [rows16, col])
              acc = acc + sv * dv
            return acc
          acc = lax.fori_loop(0, dim // 4, dquad,
                              jnp.zeros((16,), jnp.float32))
          out_v[pl.ds(cidx * LC + j * 16, 16)] = acc
          return 0
        lax.fori_loop(0, LC // 16, grp, 0)
      return 0
    lax.fori_loop(0, LCH // 2, pair, 0)

    pltpu.async_copy(out_v, dots_o.at[pl.ds(tbase, LPT)], wsem).wait()

  return pl.kernel(
      body, out_type=out_type, mesh=_mesh, scratch_types=scratch,
      compiler_params=pltpu.CompilerParams(use_tc_tiling_on_sc=False,
                                           needs_layout_passes=False))


_decode = _make_decode(OUT_DIM)


def kernel(x, edge_index, edge_label_index, W1_l, W1_r, b1, W2_l, W2_r, b2):
  x = x.astype(jnp.float32)
  src = jnp.asarray(edge_index[0], jnp.int32)
  dst = jnp.asarray(edge_index[1], jnp.int32)
  pad = jnp.zeros((NLPAD - N_LABEL,), jnp.int32)
  ls = jnp.concatenate([jnp.asarray(edge_label_index[0], jnp.int32), pad])
  ld = jnp.concatenate([jnp.asarray(edge_label_index[1], jnp.int32), pad])

  part1, cnt = _agg_cnt(x, src, dst)
  cnt2 = cnt.reshape(NC, CPAD, 1)
  h = _tc_layer(part1, cnt2, x, W1_l, W1_r, b1, relu=True)
  (part2,) = _agg_plain(h, src, dst)
  z = _tc_layer(part2, cnt2, h, W2_l, W2_r, b2, relu=False)
  (dots,) = _decode(z, ls, ld)
  return dots[:N_LABEL]
